# Initial kernel scaffold; baseline (speedup 1.0000x reference)
import functools

import jax
import jax.numpy as jnp
from jax import lax
from jax.experimental import pallas as pl
from jax.experimental.pallas import tpu as pltpu
from jax.experimental.pallas import tpu_sc as plsc

NP = 10240
BR = 1024
EP = 163840
EC = 128
TILES = 32
EPT = EP // TILES
NCH = EPT // EC
NRT = NP // 16


def _linear_body(x_ref, w_ref, b_ref, o_ref, *, act):
    y = jnp.dot(x_ref[...], w_ref[...], preferred_element_type=jnp.float32)
    y = y + b_ref[0:1, :]
    if act == "relu":
        y = jnp.maximum(y, 0.0)
    o_ref[...] = y


def tc_linear(x, wt, b, act="none"):
    n, di = x.shape
    do = wt.shape[1]
    b2d = jnp.tile(b.reshape(1, do), (8, 1))
    return pl.pallas_call(
        functools.partial(_linear_body, act=act),
        grid=(n // BR,),
        in_specs=[pl.BlockSpec((BR, di), lambda i: (i, 0)),
                  pl.BlockSpec((di, do), lambda i: (0, 0)),
                  pl.BlockSpec((8, do), lambda i: (0, 0))],
        out_specs=pl.BlockSpec((BR, do), lambda i: (i, 0)),
        out_shape=jax.ShapeDtypeStruct((n, do), jnp.float32),
    )(x, wt, b2d)


def _mlp4_body(x_ref, w1, b1, w2, b2, w3, b3, w4, b4, o_ref):
    h = jnp.maximum(jnp.dot(x_ref[...], w1[...],
                            preferred_element_type=jnp.float32) + b1[0:1, :], 0.0)
    h = jnp.maximum(jnp.dot(h, w2[...],
                            preferred_element_type=jnp.float32) + b2[0:1, :], 0.0)
    h = jnp.maximum(jnp.dot(h, w3[...],
                            preferred_element_type=jnp.float32) + b3[0:1, :], 0.0)
    o_ref[...] = jnp.dot(h, w4[...],
                         preferred_element_type=jnp.float32) + b4[0:1, :]


def tc_mlp4(x, ws, bs):
    n, di = x.shape
    do = ws[3].shape[1]
    args = [x]
    in_specs = [pl.BlockSpec((BR, di), lambda i: (i, 0))]
    for wt, b in zip(ws, bs):
        dwi, dwo = wt.shape
        args.append(wt)
        in_specs.append(pl.BlockSpec((dwi, dwo), lambda i: (0, 0)))
        args.append(jnp.tile(b.reshape(1, dwo), (8, 1)))
        in_specs.append(pl.BlockSpec((8, dwo), lambda i: (0, 0)))
    return pl.pallas_call(
        _mlp4_body,
        grid=(n // BR,),
        in_specs=in_specs,
        out_specs=pl.BlockSpec((BR, do), lambda i: (i, 0)),
        out_shape=jax.ShapeDtypeStruct((n, do), jnp.float32),
    )(*args)


def _postmp_body(s0_ref, s1_ref, deg_ref, w2_ref, b2_ref, g_ref, b_ref,
                 *rest, final):
    if final:
        f_ref, o_ref = rest
    else:
        (o_ref,) = rest
    s = s0_ref[...] + s1_ref[...]
    y = jnp.dot(s, w2_ref[...], preferred_element_type=jnp.float32)
    y = y + deg_ref[...] * b2_ref[0:1, :]
    y = jnp.maximum(y, 0.0)
    mu = jnp.mean(y, axis=1, keepdims=True)
    var = jnp.mean((y - mu) * (y - mu), axis=1, keepdims=True)
    y = (y - mu) / jnp.sqrt(var + 1e-5) * g_ref[0:1, :] + b_ref[0:1, :]
    if final:
        y = (y + f_ref[...]) * 0.5
    o_ref[...] = y


def tc_postmp(s0, s1, deg, w2t, b2, ln_g, ln_b, f=None):
    n, h2 = s0.shape
    do = w2t.shape[1]
    final = f is not None
    args = [s0, s1, deg, w2t,
            jnp.tile(b2.reshape(1, do), (8, 1)),
            jnp.tile(ln_g.reshape(1, do), (8, 1)),
            jnp.tile(ln_b.reshape(1, do), (8, 1))]
    in_specs = [pl.BlockSpec((BR, h2), lambda i: (i, 0)),
                pl.BlockSpec((BR, h2), lambda i: (i, 0)),
                pl.BlockSpec((BR, 1), lambda i: (i, 0)),
                pl.BlockSpec((h2, do), lambda i: (0, 0)),
                pl.BlockSpec((8, do), lambda i: (0, 0)),
                pl.BlockSpec((8, do), lambda i: (0, 0)),
                pl.BlockSpec((8, do), lambda i: (0, 0))]
    if final:
        args.append(f)
        in_specs.append(pl.BlockSpec((BR, do), lambda i: (i, 0)))
    return pl.pallas_call(
        functools.partial(_postmp_body, final=final),
        grid=(n // BR,),
        in_specs=in_specs,
        out_specs=pl.BlockSpec((BR, do), lambda i: (i, 0)),
        out_shape=jax.ShapeDtypeStruct((n, do), jnp.float32),
    )(*args)


def _axpy_body(x_ref, xs_ref, k_ref, o_ref):
    o_ref[...] = x_ref[...] + jnp.dot(xs_ref[...], k_ref[...],
                                      preferred_element_type=jnp.float32)


def tc_axpy(x, xs, kmat):
    n, d = x.shape
    return pl.pallas_call(
        _axpy_body,
        grid=(n // BR,),
        in_specs=[pl.BlockSpec((BR, d), lambda i: (i, 0)),
                  pl.BlockSpec((BR, d), lambda i: (i, 0)),
                  pl.BlockSpec((d, d), lambda i: (0, 0))],
        out_specs=pl.BlockSpec((BR, d), lambda i: (i, 0)),
        out_shape=jax.ShapeDtypeStruct((n, d), jnp.float32),
    )(x, xs, kmat)


def _mm_body(a_ref, b_ref, o_ref):
    o_ref[...] = jnp.dot(a_ref[...], b_ref[...],
                         preferred_element_type=jnp.float32)


def tc_mm_small(a, b):
    m, k = a.shape
    n = b.shape[1]
    return pl.pallas_call(
        _mm_body,
        in_specs=[pl.BlockSpec((m, k), lambda: (0, 0)),
                  pl.BlockSpec((k, n), lambda: (0, 0))],
        out_specs=pl.BlockSpec((m, n), lambda: (0, 0)),
        out_shape=jax.ShapeDtypeStruct((m, n), jnp.float32),
    )(a, b)


def _powers_body(k_ref, o_ref, *, nsteps):
    cur = k_ref[...]
    for i in range(nsteps):
        o_ref[i] = cur
        if i + 1 < nsteps:
            cur = jnp.dot(cur, cur, preferred_element_type=jnp.float32)


def tc_powers(kmat, nsteps):
    d = kmat.shape[0]
    return pl.pallas_call(
        functools.partial(_powers_body, nsteps=nsteps),
        in_specs=[pl.BlockSpec((d, d), lambda: (0, 0))],
        out_specs=pl.BlockSpec((nsteps, d, d), lambda: (0, 0, 0)),
        out_shape=jax.ShapeDtypeStruct((nsteps, d, d), jnp.float32),
    )(kmat)


def _sc_edge_kernel(h2):
    nvr = h2 // 16
    mesh = plsc.VectorSubcoreMesh(core_axis_name="c", subcore_axis_name="s")

    @functools.partial(
        pl.kernel,
        out_type=jax.ShapeDtypeStruct((2 * NP, h2), jnp.float32),
        mesh=mesh,
        scratch_types=[
            pltpu.VMEM((EC,), jnp.int32),
            pltpu.VMEM((EC,), jnp.int32),
            pltpu.VMEM((EC, 4), jnp.float32),
            pltpu.VMEM((EC, h2), jnp.float32),
            pltpu.VMEM((EC, h2), jnp.float32),
            pltpu.VMEM((EC, h2), jnp.float32),
            pltpu.VMEM((4, h2), jnp.float32),
            pltpu.VMEM((h2,), jnp.float32),
            pltpu.VMEM_SHARED((NP, h2), jnp.float32),
            pltpu.SemaphoreType.DMA,
            pltpu.SemaphoreType.DMA,
        ],
    )
    def k(a_hbm, b_hbm, dst_hbm, src_hbm, ea_hbm, wet_hbm, b1_hbm, out_hbm,
          dst_v, src_v, ea_v, a_v, b_v, acc_v, wet_v, b1_v, acc_sh,
          sem_a, sem_b):
        c = lax.axis_index("c")
        s = lax.axis_index("s")
        wid = c * 16 + s

        def zb(r, carry):
            for j in range(nvr):
                a_v[r, pl.ds(16 * j, 16)] = jnp.zeros((16,), jnp.float32)
            return carry
        lax.fori_loop(0, EC, zb, 0)
        for i in range(NRT // EC):
            pltpu.sync_copy(a_v, acc_sh.at[pl.ds(s * NRT + i * EC, EC)])

        pltpu.sync_copy(wet_hbm, wet_v)
        pltpu.sync_copy(b1_hbm, b1_v)
        plsc.subcore_barrier()

        def chunk(cc, carry):
            base = wid * EPT + cc * EC
            pltpu.sync_copy(dst_hbm.at[pl.ds(base, EC)], dst_v)
            pltpu.sync_copy(src_hbm.at[pl.ds(base, EC)], src_v)
            pltpu.sync_copy(ea_hbm.at[pl.ds(base, EC)], ea_v)
            cp_a = pltpu.async_copy(a_hbm.at[dst_v], a_v, sem_a)
            cp_b = pltpu.async_copy(b_hbm.at[src_v], b_v, sem_b)
            cp_a.wait()
            cp_b.wait()

            def edge(r, carry2):
                e0 = ea_v[r, 0]
                e1 = ea_v[r, 1]
                e2 = ea_v[r, 2]
                e3 = ea_v[r, 3]
                for j in range(nvr):
                    sl = pl.ds(16 * j, 16)
                    v = a_v[r, sl] + b_v[r, sl] + b1_v[sl]
                    v = v + e0 * wet_v[0, sl] + e1 * wet_v[1, sl]
                    v = v + e2 * wet_v[2, sl] + e3 * wet_v[3, sl]
                    acc_v[r, sl] = jnp.maximum(v, 0.0)
                return carry2
            lax.fori_loop(0, EC, edge, 0)
            pltpu.sync_copy(acc_v, acc_sh.at[dst_v], add=True)
            return carry
        lax.fori_loop(0, NCH, chunk, 0)

        plsc.subcore_barrier()
        for i in range(NRT // EC):
            off = s * NRT + i * EC
            pltpu.sync_copy(acc_sh.at[pl.ds(off, EC)], a_v)
            pltpu.sync_copy(a_v, out_hbm.at[pl.ds(c * NP + off, EC)])

    return k


def _sc_deg_kernel():
    mesh = plsc.VectorSubcoreMesh(core_axis_name="c", subcore_axis_name="s")

    @functools.partial(
        pl.kernel,
        out_type=jax.ShapeDtypeStruct((2 * NP, 16), jnp.float32),
        mesh=mesh,
        scratch_types=[
            pltpu.VMEM((EC,), jnp.int32),
            pltpu.VMEM((EC, 16), jnp.float32),
            pltpu.VMEM((EC, 16), jnp.float32),
            pltpu.VMEM_SHARED((NP, 16), jnp.float32),
        ],
    )
    def k(dst_hbm, out_hbm, dst_v, ones_v, zero_v, acc_sh):
        c = lax.axis_index("c")
        s = lax.axis_index("s")
        wid = c * 16 + s

        one_row = jnp.zeros((16,), jnp.float32).at[0].set(1.0)

        def fill(r, carry):
            ones_v[r, pl.ds(0, 16)] = one_row
            zero_v[r, pl.ds(0, 16)] = jnp.zeros((16,), jnp.float32)
            return carry
        lax.fori_loop(0, EC, fill, 0)
        for i in range(NRT // EC):
            pltpu.sync_copy(zero_v, acc_sh.at[pl.ds(s * NRT + i * EC, EC)])
        plsc.subcore_barrier()

        def chunk(cc, carry):
            base = wid * EPT + cc * EC
            pltpu.sync_copy(dst_hbm.at[pl.ds(base, EC)], dst_v)
            pltpu.sync_copy(ones_v, acc_sh.at[dst_v], add=True)
            return carry
        lax.fori_loop(0, NCH, chunk, 0)

        plsc.subcore_barrier()
        for i in range(NRT // EC):
            off = s * NRT + i * EC
            pltpu.sync_copy(acc_sh.at[pl.ds(off, EC)], zero_v)
            pltpu.sync_copy(zero_v, out_hbm.at[pl.ds(c * NP + off, EC)])

    return k


def _gnn_fast(p, x_pad, dsti, srci, ea_pad, deg, in_dim, hid, out_dim):
    dims = [(in_dim, hid, "conv1", "norm1"),
            (hid, hid // 2, "conv2", "norm2"),
            (hid // 2, out_dim, "conv3", "norm3")]
    f = tc_mlp4(
        x_pad,
        [p["fc1"]["w"].T, p["fc2"]["w"].T, p["fc3"]["w"].T, p["fc4"]["w"].T],
        [p["fc1"]["b"], p["fc2"]["b"], p["fc3"]["b"], p["fc4"]["b"]])
    h = x_pad
    for li, (di, h2, cname, nname) in enumerate(dims):
        l1 = p[cname + "_l1"]
        l2 = p[cname + "_l2"]
        w1 = l1["w"]
        wd = w1[:, :di].T
        ws = w1[:, di:2 * di].T
        we = w1[:, 2 * di:].T
        a = tc_linear(h, wd, jnp.zeros((h2,), jnp.float32))
        b = tc_linear(h, ws, jnp.zeros((h2,), jnp.float32))
        s_parts = _sc_edge_kernel(h2)(a, b, dsti, srci, ea_pad,
                                      we, l1["b"])
        s0 = s_parts[:NP]
        s1 = s_parts[NP:]
        nrm = p[nname]
        is_last = li == 2
        h = tc_postmp(s0, s1, deg, l2["w"].T, l2["b"], nrm["g"], nrm["b"],
                      f=f if is_last else None)
    return h


def kernel(x, edge_index, edge_attr, enc_params, dec_params, koopman_blocks,
           sigma, L_w):
    n, in_dim = x.shape
    e = edge_attr.shape[0]
    hid = enc_params["fc1"]["w"].shape[0]
    koop = enc_params["fc4"]["w"].shape[0]
    num_obj, _, hh = sigma.shape
    m = koopman_blocks.shape[1]

    x_pad = jnp.zeros((NP, in_dim), jnp.float32).at[:n].set(x)
    dsti = jnp.full((EP,), NP - 1, jnp.int32).at[:e].set(
        edge_index[1].astype(jnp.int32))
    srci = jnp.zeros((EP,), jnp.int32).at[:e].set(
        edge_index[0].astype(jnp.int32))
    ea_pad = jnp.zeros((EP, 4), jnp.float32).at[:e].set(edge_attr)

    deg_parts = _sc_deg_kernel()(dsti)
    deg = (deg_parts[:NP, 0] + deg_parts[NP:, 0]).reshape(NP, 1)

    ks_pad = _gnn_fast(enc_params, x_pad, dsti, srci, ea_pad, deg,
                       in_dim, hid, koop)

    dec_ae_pad = _gnn_fast(dec_params, ks_pad, dsti, srci, ea_pad, deg,
                           koop, hid, in_dim)

    sigma2 = sigma.reshape(num_obj * num_obj, hh)
    koop2 = koopman_blocks.reshape(hh, m * m)
    kb2 = tc_mm_small(sigma2, koop2)
    kmat = kb2.reshape(num_obj, num_obj, m, m).transpose(0, 2, 1, 3)
    kmat = kmat.reshape(num_obj * m, num_obj * m)

    t_len = n
    u_pad = jnp.zeros((NP, 4), jnp.float32).at[:t_len - 1].set(
        edge_attr[:t_len - 1])
    hb = tc_linear(u_pad, L_w.T, jnp.zeros((koop,), jnp.float32))
    g0 = ks_pad[0]
    xs0 = jnp.concatenate([g0[None, :], hb[:NP - 1]], axis=0)

    nsteps = 1
    while (1 << nsteps) < t_len:
        nsteps += 1
    kpow = tc_powers(kmat, nsteps)

    xcur = xs0
    for kstep in range(nsteps):
        sft = 1 << kstep
        shifted = jnp.concatenate(
            [jnp.zeros((sft, koop), jnp.float32), xcur[:NP - sft]], axis=0)
        xcur = tc_axpy(xcur, shifted, kpow[kstep])

    g_hat_pad = jnp.concatenate(
        [xcur[:t_len], jnp.zeros((NP - t_len, koop), jnp.float32)], axis=0)

    dec_ro_pad = _gnn_fast(dec_params, g_hat_pad, dsti, srci, ea_pad, deg,
                           koop, hid, in_dim)

    return (dec_ae_pad[:n], dec_ro_pad[:n], ks_pad[:n])


# speedup vs baseline: 2.8087x; 2.8087x over previous
import functools

import jax
import jax.numpy as jnp
from jax import lax
from jax.experimental import pallas as pl
from jax.experimental.pallas import tpu as pltpu
from jax.experimental.pallas import tpu_sc as plsc

_PH = jax.lax.Precision.HIGHEST

NP = 10240
BR = 1024
EP = 163840
EC = 128
TILES = 32
EPT = EP // TILES
NCH = EPT // EC
NRT = NP // 16


def _linear_body(x_ref, w_ref, b_ref, o_ref, *, act):
    y = jnp.dot(x_ref[...], w_ref[...], preferred_element_type=jnp.float32, precision=_PH)
    y = y + b_ref[0:1, :]
    if act == "relu":
        y = jnp.maximum(y, 0.0)
    o_ref[...] = y


def _linear_bf16_body(x_ref, w_ref, b_ref, o_ref):
    xb = x_ref[...].astype(jnp.bfloat16)
    wb = w_ref[...].astype(jnp.bfloat16)
    o_ref[...] = jnp.dot(xb, wb, preferred_element_type=jnp.float32) + b_ref[0:1, :]


def tc_linear_bf16(x, wt, b):
    n, di = x.shape
    do = wt.shape[1]
    b2d = jnp.tile(b.reshape(1, do), (8, 1))
    return pl.pallas_call(
        _linear_bf16_body,
        grid=(n // BR,),
        in_specs=[pl.BlockSpec((BR, di), lambda i: (i, 0)),
                  pl.BlockSpec((di, do), lambda i: (0, 0)),
                  pl.BlockSpec((8, do), lambda i: (0, 0))],
        out_specs=pl.BlockSpec((BR, do), lambda i: (i, 0)),
        out_shape=jax.ShapeDtypeStruct((n, do), jnp.float32),
    )(x, wt, b2d)


def tc_linear(x, wt, b, act="none"):
    n, di = x.shape
    do = wt.shape[1]
    b2d = jnp.tile(b.reshape(1, do), (8, 1))
    return pl.pallas_call(
        functools.partial(_linear_body, act=act),
        grid=(n // BR,),
        in_specs=[pl.BlockSpec((BR, di), lambda i: (i, 0)),
                  pl.BlockSpec((di, do), lambda i: (0, 0)),
                  pl.BlockSpec((8, do), lambda i: (0, 0))],
        out_specs=pl.BlockSpec((BR, do), lambda i: (i, 0)),
        out_shape=jax.ShapeDtypeStruct((n, do), jnp.float32),
    )(x, wt, b2d)


def _mlp4_body(x_ref, w1, b1, w2, b2, w3, b3, w4, b4, o_ref):
    h = jnp.maximum(jnp.dot(x_ref[...], w1[...],
                            preferred_element_type=jnp.float32, precision=_PH) + b1[0:1, :], 0.0)
    h = jnp.maximum(jnp.dot(h, w2[...],
                            preferred_element_type=jnp.float32, precision=_PH) + b2[0:1, :], 0.0)
    h = jnp.maximum(jnp.dot(h, w3[...],
                            preferred_element_type=jnp.float32, precision=_PH) + b3[0:1, :], 0.0)
    o_ref[...] = jnp.dot(h, w4[...],
                         preferred_element_type=jnp.float32, precision=_PH) + b4[0:1, :]


def tc_mlp4(x, ws, bs):
    n, di = x.shape
    do = ws[3].shape[1]
    args = [x]
    in_specs = [pl.BlockSpec((BR, di), lambda i: (i, 0))]
    for wt, b in zip(ws, bs):
        dwi, dwo = wt.shape
        args.append(wt)
        in_specs.append(pl.BlockSpec((dwi, dwo), lambda i: (0, 0)))
        args.append(jnp.tile(b.reshape(1, dwo), (8, 1)))
        in_specs.append(pl.BlockSpec((8, dwo), lambda i: (0, 0)))
    return pl.pallas_call(
        _mlp4_body,
        grid=(n // BR,),
        in_specs=in_specs,
        out_specs=pl.BlockSpec((BR, do), lambda i: (i, 0)),
        out_shape=jax.ShapeDtypeStruct((n, do), jnp.float32),
    )(*args)


def _postmp_body(s0_ref, s1_ref, deg_ref, w2_ref, b2_ref, g_ref, b_ref,
                 *rest, final, h2):
    if final:
        f_ref, o_ref = rest
    else:
        (o_ref,) = rest
    s = s0_ref[..., :h2] + s1_ref[..., :h2]
    y = jnp.dot(s, w2_ref[...], preferred_element_type=jnp.float32, precision=_PH)
    y = y + deg_ref[...] * b2_ref[0:1, :]
    y = jnp.maximum(y, 0.0)
    mu = jnp.mean(y, axis=1, keepdims=True)
    var = jnp.mean((y - mu) * (y - mu), axis=1, keepdims=True)
    y = (y - mu) / jnp.sqrt(var + 1e-5) * g_ref[0:1, :] + b_ref[0:1, :]
    if final:
        y = (y + f_ref[...]) * 0.5
    o_ref[...] = y


def tc_postmp(s0, s1, deg, h2, w2t, b2, ln_g, ln_b, f=None):
    n, wpad = s0.shape
    do = w2t.shape[1]
    final = f is not None
    args = [s0, s1, deg, w2t,
            jnp.tile(b2.reshape(1, do), (8, 1)),
            jnp.tile(ln_g.reshape(1, do), (8, 1)),
            jnp.tile(ln_b.reshape(1, do), (8, 1))]
    in_specs = [pl.BlockSpec((BR, wpad), lambda i: (i, 0)),
                pl.BlockSpec((BR, wpad), lambda i: (i, 0)),
                pl.BlockSpec((BR, 1), lambda i: (i, 0)),
                pl.BlockSpec((h2, do), lambda i: (0, 0)),
                pl.BlockSpec((8, do), lambda i: (0, 0)),
                pl.BlockSpec((8, do), lambda i: (0, 0)),
                pl.BlockSpec((8, do), lambda i: (0, 0))]
    if final:
        args.append(f)
        in_specs.append(pl.BlockSpec((BR, do), lambda i: (i, 0)))
    return pl.pallas_call(
        functools.partial(_postmp_body, final=final, h2=h2),
        grid=(n // BR,),
        in_specs=in_specs,
        out_specs=pl.BlockSpec((BR, do), lambda i: (i, 0)),
        out_shape=jax.ShapeDtypeStruct((n, do), jnp.float32),
    )(*args)


def _mm_body(a_ref, b_ref, o_ref):
    o_ref[...] = jnp.dot(a_ref[...], b_ref[...],
                         preferred_element_type=jnp.float32, precision=_PH)


def tc_mm_small(a, b):
    m, k = a.shape
    n = b.shape[1]
    return pl.pallas_call(
        _mm_body,
        in_specs=[pl.BlockSpec((m, k), lambda: (0, 0)),
                  pl.BlockSpec((k, n), lambda: (0, 0))],
        out_specs=pl.BlockSpec((m, n), lambda: (0, 0)),
        out_shape=jax.ShapeDtypeStruct((m, n), jnp.float32),
    )(a, b)


def _scan_seq_body(g0_ref, c_ref, kb_ref, o_ref, *, t_len):
    kbv = kb_ref[...]
    g = g0_ref[0:1, :]
    o_ref[0:1, :] = g

    def step(t, g):
        gb = g.astype(jnp.bfloat16)
        ng = jnp.dot(gb, kbv, preferred_element_type=jnp.float32)
        ng = ng + c_ref[pl.ds(t, 1), :]
        o_ref[pl.ds(t + 1, 1), :] = ng
        return ng
    lax.fori_loop(0, t_len - 1, step, g)


def tc_scan_seq(g0row, c, kmat_bf16, t_len):
    d = c.shape[1]
    return pl.pallas_call(
        functools.partial(_scan_seq_body, t_len=t_len),
        in_specs=[pl.BlockSpec((8, d), lambda: (0, 0)),
                  pl.BlockSpec(c.shape, lambda: (0, 0)),
                  pl.BlockSpec((d, d), lambda: (0, 0))],
        out_specs=pl.BlockSpec(c.shape, lambda: (0, 0)),
        out_shape=jax.ShapeDtypeStruct(c.shape, jnp.float32),
    )(g0row, c, kmat_bf16)


H2P = 128


def _sc_edge_kernel(h2):
    nvr = h2 // 16
    mesh = plsc.VectorSubcoreMesh(core_axis_name="c", subcore_axis_name="s")

    @functools.partial(
        pl.kernel,
        out_type=jax.ShapeDtypeStruct((2 * NP, h2), jnp.float32),
        mesh=mesh,
        scratch_types=[
            pltpu.VMEM((EC,), jnp.int32),
            pltpu.VMEM((EC,), jnp.int32),
            pltpu.VMEM((EC * 4 + 16,), jnp.float32),
            pltpu.VMEM((EC, h2), jnp.float32),
            pltpu.VMEM((EC, h2), jnp.float32),
            pltpu.VMEM((4, h2), jnp.float32),
            pltpu.VMEM((h2,), jnp.float32),
            pltpu.VMEM_SHARED((NP, h2), jnp.float32),
            pltpu.SemaphoreType.DMA,
            pltpu.SemaphoreType.DMA,
        ],
    )
    def k(a_hbm, b_hbm, dst_hbm, src_hbm, ea_hbm, wet_hbm, b1_hbm, out_hbm,
          dst_v, src_v, ea_v, a_v, b_v, wet_v, b1_v, acc_sh,
          sem_a, sem_b):
        c = lax.axis_index("c")
        s = lax.axis_index("s")
        wid = c * 16 + s

        def zb(r, carry):
            for j in range(nvr):
                a_v[r, pl.ds(16 * j, 16)] = jnp.zeros((16,), jnp.float32)
            return carry
        lax.fori_loop(0, EC, zb, 0)
        for i in range(NRT // EC):
            pltpu.sync_copy(a_v, acc_sh.at[pl.ds(s * NRT + i * EC, EC)])

        pltpu.sync_copy(wet_hbm, wet_v)
        pltpu.sync_copy(b1_hbm, b1_v)
        plsc.subcore_barrier()

        def chunk(cc, carry):
            base = wid * EPT + cc * EC
            pltpu.sync_copy(dst_hbm.at[pl.ds(base, EC)], dst_v)
            pltpu.sync_copy(src_hbm.at[pl.ds(base, EC)], src_v)
            pltpu.sync_copy(ea_hbm.at[pl.ds(base * 4, EC * 4)],
                            ea_v.at[pl.ds(0, EC * 4)])
            cp_a = pltpu.async_copy(a_hbm.at[dst_v], a_v, sem_a)
            cp_b = pltpu.async_copy(b_hbm.at[src_v], b_v, sem_b)
            cp_a.wait()
            cp_b.wait()

            def edge(r, carry2):
                ev = ea_v[pl.ds(4 * r, 16)]
                e0 = ev[0]
                e1 = ev[1]
                e2 = ev[2]
                e3 = ev[3]
                for j in range(nvr):
                    sl = pl.ds(16 * j, 16)
                    v = a_v[r, sl] + b_v[r, sl] + b1_v[sl]
                    v = v + e0 * wet_v[0, sl] + e1 * wet_v[1, sl]
                    v = v + e2 * wet_v[2, sl] + e3 * wet_v[3, sl]
                    a_v[r, sl] = jnp.maximum(v, 0.0)
                return carry2
            lax.fori_loop(0, EC, edge, 0)
            pltpu.sync_copy(a_v, acc_sh.at[dst_v], add=True)
            return carry
        lax.fori_loop(0, NCH, chunk, 0)

        plsc.subcore_barrier()
        for i in range(NRT // EC):
            off = s * NRT + i * EC
            pltpu.sync_copy(acc_sh.at[pl.ds(off, EC)], a_v)
            pltpu.sync_copy(a_v, out_hbm.at[pl.ds(c * NP + off, EC)])

    return k


def _sc_deg_kernel():
    mesh = plsc.VectorSubcoreMesh(core_axis_name="c", subcore_axis_name="s")

    @functools.partial(
        pl.kernel,
        out_type=jax.ShapeDtypeStruct((2 * NP, 128), jnp.float32),
        mesh=mesh,
        scratch_types=[
            pltpu.VMEM((EC,), jnp.int32),
            pltpu.VMEM((EC, 128), jnp.float32),
            pltpu.VMEM((EC, 128), jnp.float32),
            pltpu.VMEM_SHARED((NP, 128), jnp.float32),
        ],
    )
    def k(dst_hbm, out_hbm, dst_v, ones_v, zero_v, acc_sh):
        c = lax.axis_index("c")
        s = lax.axis_index("s")
        wid = c * 16 + s

        one_row = jnp.where(lax.iota(jnp.int32, 16) == 0,
                            jnp.float32(1.0), jnp.float32(0.0))

        def fill(r, carry):
            ones_v[r, pl.ds(0, 16)] = one_row
            for j in range(1, 8):
                ones_v[r, pl.ds(16 * j, 16)] = jnp.zeros((16,), jnp.float32)
            for j in range(8):
                zero_v[r, pl.ds(16 * j, 16)] = jnp.zeros((16,), jnp.float32)
            return carry
        lax.fori_loop(0, EC, fill, 0)
        for i in range(NRT // EC):
            pltpu.sync_copy(zero_v, acc_sh.at[pl.ds(s * NRT + i * EC, EC)])
        plsc.subcore_barrier()

        def chunk(cc, carry):
            base = wid * EPT + cc * EC
            pltpu.sync_copy(dst_hbm.at[pl.ds(base, EC)], dst_v)
            pltpu.sync_copy(ones_v, acc_sh.at[dst_v], add=True)
            return carry
        lax.fori_loop(0, NCH, chunk, 0)

        plsc.subcore_barrier()
        for i in range(NRT // EC):
            off = s * NRT + i * EC
            pltpu.sync_copy(acc_sh.at[pl.ds(off, EC)], zero_v)
            pltpu.sync_copy(zero_v, out_hbm.at[pl.ds(c * NP + off, EC)])

    return k


_EDGE_KERNEL = _sc_edge_kernel(H2P)
_DEG_KERNEL = _sc_deg_kernel()


def _gnn_fast(p, x_pad, dsti, srci, ea_pad, deg, in_dim, hid, out_dim):
    dims = [(in_dim, hid, "conv1", "norm1"),
            (hid, hid // 2, "conv2", "norm2"),
            (hid // 2, out_dim, "conv3", "norm3")]
    f = tc_mlp4(
        x_pad,
        [p["fc1"]["w"].T, p["fc2"]["w"].T, p["fc3"]["w"].T, p["fc4"]["w"].T],
        [p["fc1"]["b"], p["fc2"]["b"], p["fc3"]["b"], p["fc4"]["b"]])
    h = x_pad
    for li, (di, h2, cname, nname) in enumerate(dims):
        l1 = p[cname + "_l1"]
        l2 = p[cname + "_l2"]
        w1 = l1["w"]
        pad = jnp.zeros((H2P, di), jnp.float32).at[:h2].set
        wd = pad(w1[:, :di]).T
        ws = pad(w1[:, di:2 * di]).T
        we = jnp.zeros((4, H2P), jnp.float32).at[:, :h2].set(w1[:, 2 * di:].T)
        b1 = jnp.zeros((H2P,), jnp.float32).at[:h2].set(l1["b"])
        a = tc_linear(h, wd, jnp.zeros((H2P,), jnp.float32))
        b = tc_linear(h, ws, jnp.zeros((H2P,), jnp.float32))
        s_parts = _EDGE_KERNEL(a, b, dsti, srci, ea_pad, we, b1)
        s0 = s_parts[:NP]
        s1 = s_parts[NP:]
        nrm = p[nname]
        is_last = li == 2
        h = tc_postmp(s0, s1, deg, h2, l2["w"].T, l2["b"], nrm["g"],
                      nrm["b"], f=f if is_last else None)
    return h


def kernel(x, edge_index, edge_attr, enc_params, dec_params, koopman_blocks,
           sigma, L_w):
    n, in_dim = x.shape
    e = edge_attr.shape[0]
    hid = enc_params["fc1"]["w"].shape[0]
    koop = enc_params["fc4"]["w"].shape[0]
    num_obj, _, hh = sigma.shape
    m = koopman_blocks.shape[1]

    x_pad = jnp.zeros((NP, in_dim), jnp.float32).at[:n].set(x)
    dsti = jnp.full((EP,), NP - 1, jnp.int32).at[:e].set(
        edge_index[1].astype(jnp.int32))
    srci = jnp.zeros((EP,), jnp.int32).at[:e].set(
        edge_index[0].astype(jnp.int32))
    ea_pad = jnp.zeros((EP, 4), jnp.float32).at[:e].set(edge_attr)
    ea_pad = ea_pad.reshape(EP * 4)

    deg_parts = _DEG_KERNEL(dsti)
    deg = (deg_parts[:NP, 0] + deg_parts[NP:, 0]).reshape(NP, 1)

    ks_pad = _gnn_fast(enc_params, x_pad, dsti, srci, ea_pad, deg,
                       in_dim, hid, koop)

    dec_ae_pad = _gnn_fast(dec_params, ks_pad, dsti, srci, ea_pad, deg,
                           koop, hid, in_dim)

    sigma2 = sigma.reshape(num_obj * num_obj, hh)
    koop2 = koopman_blocks.reshape(hh, m * m)
    kb2 = tc_mm_small(sigma2, koop2)
    kmat = kb2.reshape(num_obj, num_obj, m, m).transpose(0, 2, 1, 3)
    kmat = kmat.reshape(num_obj * m, num_obj * m)

    t_len = n
    u_pad = jnp.zeros((NP, 4), jnp.float32).at[:t_len - 1].set(
        edge_attr[:t_len - 1])
    c_in = tc_linear_bf16(u_pad, L_w.T, jnp.zeros((koop,), jnp.float32))
    gs = tc_scan_seq(ks_pad[0:8], c_in, kmat.astype(jnp.bfloat16), t_len)
    g_hat_pad = jnp.concatenate(
        [gs[:t_len], jnp.zeros((NP - t_len, koop), jnp.float32)], axis=0)

    dec_ro_pad = _gnn_fast(dec_params, g_hat_pad, dsti, srci, ea_pad, deg,
                           koop, hid, in_dim)

    return (dec_ae_pad[:n], dec_ro_pad[:n], ks_pad[:n])


# speedup vs baseline: 5.5749x; 1.9848x over previous
import functools

import jax
import jax.numpy as jnp
from jax import lax
from jax.experimental import pallas as pl
from jax.experimental.pallas import tpu as pltpu
from jax.experimental.pallas import tpu_sc as plsc

_PH = jax.lax.Precision.HIGHEST

NP = 10240
BR = 1024
EP = 163840
EC = 64
TILES = 32
EPT = EP // TILES
NCH = EPT // EC
NRT = NP // 16


def _linear_body(x_ref, w_ref, b_ref, o_ref, *, act):
    y = jnp.dot(x_ref[...], w_ref[...], preferred_element_type=jnp.float32, precision=_PH)
    y = y + b_ref[0:1, :]
    if act == "relu":
        y = jnp.maximum(y, 0.0)
    o_ref[...] = y


def _linear_bf16_body(x_ref, w_ref, b_ref, o_ref):
    xb = x_ref[...].astype(jnp.bfloat16)
    wb = w_ref[...].astype(jnp.bfloat16)
    o_ref[...] = jnp.dot(xb, wb, preferred_element_type=jnp.float32) + b_ref[0:1, :]


def tc_linear_bf16(x, wt, b):
    n, di = x.shape
    do = wt.shape[1]
    b2d = jnp.tile(b.reshape(1, do), (8, 1))
    return pl.pallas_call(
        _linear_bf16_body,
        grid=(n // BR,),
        in_specs=[pl.BlockSpec((BR, di), lambda i: (i, 0)),
                  pl.BlockSpec((di, do), lambda i: (0, 0)),
                  pl.BlockSpec((8, do), lambda i: (0, 0))],
        out_specs=pl.BlockSpec((BR, do), lambda i: (i, 0)),
        out_shape=jax.ShapeDtypeStruct((n, do), jnp.float32),
    )(x, wt, b2d)


def tc_linear(x, wt, b, act="none"):
    n, di = x.shape
    do = wt.shape[1]
    b2d = jnp.tile(b.reshape(1, do), (8, 1))
    return pl.pallas_call(
        functools.partial(_linear_body, act=act),
        grid=(n // BR,),
        in_specs=[pl.BlockSpec((BR, di), lambda i: (i, 0)),
                  pl.BlockSpec((di, do), lambda i: (0, 0)),
                  pl.BlockSpec((8, do), lambda i: (0, 0))],
        out_specs=pl.BlockSpec((BR, do), lambda i: (i, 0)),
        out_shape=jax.ShapeDtypeStruct((n, do), jnp.float32),
    )(x, wt, b2d)


def _mlp4_body(x_ref, w1, b1, w2, b2, w3, b3, w4, b4, o_ref):
    h = jnp.maximum(jnp.dot(x_ref[...], w1[...],
                            preferred_element_type=jnp.float32, precision=_PH) + b1[0:1, :], 0.0)
    h = jnp.maximum(jnp.dot(h, w2[...],
                            preferred_element_type=jnp.float32, precision=_PH) + b2[0:1, :], 0.0)
    h = jnp.maximum(jnp.dot(h, w3[...],
                            preferred_element_type=jnp.float32, precision=_PH) + b3[0:1, :], 0.0)
    o_ref[...] = jnp.dot(h, w4[...],
                         preferred_element_type=jnp.float32, precision=_PH) + b4[0:1, :]


def tc_mlp4(x, ws, bs):
    n, di = x.shape
    do = ws[3].shape[1]
    args = [x]
    in_specs = [pl.BlockSpec((BR, di), lambda i: (i, 0))]
    for wt, b in zip(ws, bs):
        dwi, dwo = wt.shape
        args.append(wt)
        in_specs.append(pl.BlockSpec((dwi, dwo), lambda i: (0, 0)))
        args.append(jnp.tile(b.reshape(1, dwo), (8, 1)))
        in_specs.append(pl.BlockSpec((8, dwo), lambda i: (0, 0)))
    return pl.pallas_call(
        _mlp4_body,
        grid=(n // BR,),
        in_specs=in_specs,
        out_specs=pl.BlockSpec((BR, do), lambda i: (i, 0)),
        out_shape=jax.ShapeDtypeStruct((n, do), jnp.float32),
    )(*args)


def _postmp_body(s0_ref, s1_ref, deg_ref, w2_ref, b2_ref, g_ref, b_ref,
                 *rest, final, h2):
    if final:
        f_ref, o_ref = rest
    else:
        (o_ref,) = rest
    s = s0_ref[..., :h2] + s1_ref[..., :h2]
    y = jnp.dot(s, w2_ref[...], preferred_element_type=jnp.float32, precision=_PH)
    y = y + deg_ref[...] * b2_ref[0:1, :]
    y = jnp.maximum(y, 0.0)
    mu = jnp.mean(y, axis=1, keepdims=True)
    var = jnp.mean((y - mu) * (y - mu), axis=1, keepdims=True)
    y = (y - mu) / jnp.sqrt(var + 1e-5) * g_ref[0:1, :] + b_ref[0:1, :]
    if final:
        y = (y + f_ref[...]) * 0.5
    o_ref[...] = y


def tc_postmp(s0, s1, deg, h2, w2t, b2, ln_g, ln_b, f=None):
    n, wpad = s0.shape
    do = w2t.shape[1]
    final = f is not None
    args = [s0, s1, deg, w2t,
            jnp.tile(b2.reshape(1, do), (8, 1)),
            jnp.tile(ln_g.reshape(1, do), (8, 1)),
            jnp.tile(ln_b.reshape(1, do), (8, 1))]
    in_specs = [pl.BlockSpec((BR, wpad), lambda i: (i, 0)),
                pl.BlockSpec((BR, wpad), lambda i: (i, 0)),
                pl.BlockSpec((BR, 1), lambda i: (i, 0)),
                pl.BlockSpec((h2, do), lambda i: (0, 0)),
                pl.BlockSpec((8, do), lambda i: (0, 0)),
                pl.BlockSpec((8, do), lambda i: (0, 0)),
                pl.BlockSpec((8, do), lambda i: (0, 0))]
    if final:
        args.append(f)
        in_specs.append(pl.BlockSpec((BR, do), lambda i: (i, 0)))
    return pl.pallas_call(
        functools.partial(_postmp_body, final=final, h2=h2),
        grid=(n // BR,),
        in_specs=in_specs,
        out_specs=pl.BlockSpec((BR, do), lambda i: (i, 0)),
        out_shape=jax.ShapeDtypeStruct((n, do), jnp.float32),
    )(*args)


def _mm_body(a_ref, b_ref, o_ref):
    o_ref[...] = jnp.dot(a_ref[...], b_ref[...],
                         preferred_element_type=jnp.float32, precision=_PH)


def tc_mm_small(a, b):
    m, k = a.shape
    n = b.shape[1]
    return pl.pallas_call(
        _mm_body,
        in_specs=[pl.BlockSpec((m, k), lambda: (0, 0)),
                  pl.BlockSpec((k, n), lambda: (0, 0))],
        out_specs=pl.BlockSpec((m, n), lambda: (0, 0)),
        out_shape=jax.ShapeDtypeStruct((m, n), jnp.float32),
    )(a, b)


def _scan_seq_body(g0_ref, c_ref, kb_ref, o_ref, *, t_len):
    kbv = kb_ref[...]
    g = g0_ref[0:1, :]
    o_ref[0:1, :] = g

    def step(t, g):
        gb = g.astype(jnp.bfloat16)
        ng = jnp.dot(gb, kbv, preferred_element_type=jnp.float32)
        ng = ng + c_ref[pl.ds(t, 1), :]
        o_ref[pl.ds(t + 1, 1), :] = ng
        return ng
    lax.fori_loop(0, t_len - 1, step, g)


def tc_scan_seq(g0row, c, kmat_bf16, t_len):
    d = c.shape[1]
    return pl.pallas_call(
        functools.partial(_scan_seq_body, t_len=t_len),
        in_specs=[pl.BlockSpec((8, d), lambda: (0, 0)),
                  pl.BlockSpec(c.shape, lambda: (0, 0)),
                  pl.BlockSpec((d, d), lambda: (0, 0))],
        out_specs=pl.BlockSpec(c.shape, lambda: (0, 0)),
        out_shape=jax.ShapeDtypeStruct(c.shape, jnp.float32),
    )(g0row, c, kmat_bf16)


H2P = 128


def _sc_edge_kernel(h2):
    nvr = h2 // 16
    mesh = plsc.VectorSubcoreMesh(core_axis_name="c", subcore_axis_name="s")

    @functools.partial(
        pl.kernel,
        out_type=jax.ShapeDtypeStruct((2 * NP, h2), jnp.float32),
        mesh=mesh,
        scratch_types=[
            pltpu.VMEM((2, EC), jnp.int32),
            pltpu.VMEM((2, EC), jnp.int32),
            pltpu.VMEM((EC * 4 + 16,), jnp.float32),
            pltpu.VMEM((EC * 4 + 16,), jnp.float32),
            pltpu.VMEM((EC, h2), jnp.float32),
            pltpu.VMEM((EC, h2), jnp.float32),
            pltpu.VMEM((EC, h2), jnp.float32),
            pltpu.VMEM((EC, h2), jnp.float32),
            pltpu.VMEM((4, h2), jnp.float32),
            pltpu.VMEM((h2,), jnp.float32),
            pltpu.VMEM_SHARED((NP, h2), jnp.float32),
            pltpu.SemaphoreType.DMA,
            pltpu.SemaphoreType.DMA,
            pltpu.SemaphoreType.DMA,
            pltpu.SemaphoreType.DMA,
            pltpu.SemaphoreType.DMA,
            pltpu.SemaphoreType.DMA,
        ],
    )
    def k(a_hbm, b_hbm, eidx_hbm, ea_hbm, wet_hbm, b1_hbm, out_hbm,
          idx0, idx1, ea0, ea1, av0, av1, bv0, bv1, wet_v, bias_v,
          acc_sh, se0, se1, sa0, sa1, sb0, sb1):
        c = lax.axis_index("c")
        s = lax.axis_index("s")
        wid = c * 16 + s
        idxs, eas, avs, bvs = [idx0, idx1], [ea0, ea1], [av0, av1], [bv0, bv1]
        sems_e, sems_a, sems_b = [se0, se1], [sa0, sa1], [sb0, sb1]

        pltpu.sync_copy(wet_hbm, wet_v)
        pltpu.sync_copy(b1_hbm, bias_v)

        def zb(r, carry):
            for j in range(nvr):
                av0[r, pl.ds(16 * j, 16)] = jnp.zeros((16,), jnp.float32)
            return carry
        lax.fori_loop(0, EC, zb, 0)
        for i in range(NRT // EC):
            pltpu.sync_copy(av0, acc_sh.at[pl.ds(s * NRT + i * EC, EC)])
        plsc.subcore_barrier()

        wvals = [[wet_v[d, pl.ds(16 * j, 16)] for j in range(nvr)]
                 for d in range(4)]
        bvals = [bias_v[pl.ds(16 * j, 16)] for j in range(nvr)]

        def fetch(cc, k):
            gcc = wid * NCH + cc
            base4 = (wid * EPT + cc * EC) * 4
            pltpu.sync_copy(eidx_hbm.at[gcc], idxs[k])
            pltpu.async_copy(ea_hbm.at[pl.ds(base4, EC * 4)],
                             eas[k].at[pl.ds(0, EC * 4)], sems_e[k])
            pltpu.async_copy(a_hbm.at[idxs[k].at[0]], avs[k], sems_a[k])
            pltpu.async_copy(b_hbm.at[idxs[k].at[1]], bvs[k], sems_b[k])

        def compute_scatter(cc, k):
            av, bv, eav = avs[k], bvs[k], eas[k]
            base4 = (wid * EPT + cc * EC) * 4
            pltpu.make_async_copy(ea_hbm.at[pl.ds(base4, EC * 4)],
                                  eav.at[pl.ds(0, EC * 4)], sems_e[k]).wait()
            pltpu.make_async_copy(a_hbm.at[idxs[k].at[0]], av,
                                  sems_a[k]).wait()
            pltpu.make_async_copy(b_hbm.at[idxs[k].at[1]], bv,
                                  sems_b[k]).wait()

            def edge(r, carry):
                ev = eav[pl.ds(4 * r, 16)]
                e0, e1, e2, e3 = ev[0], ev[1], ev[2], ev[3]
                for j in range(nvr):
                    sl = pl.ds(16 * j, 16)
                    v = av[r, sl] + bv[r, sl] + bvals[j]
                    v = v + e0 * wvals[0][j] + e1 * wvals[1][j]
                    v = v + e2 * wvals[2][j] + e3 * wvals[3][j]
                    av[r, sl] = jnp.maximum(v, 0.0)
                return carry
            lax.fori_loop(0, EC, edge, 0)
            pltpu.sync_copy(av, acc_sh.at[idxs[k].at[0]], add=True)

        fetch(0, 0)

        def pair(jj, carry):
            cc0 = 2 * jj
            fetch(cc0 + 1, 1)
            compute_scatter(cc0, 0)

            @pl.when(cc0 + 2 < NCH)
            def _():
                fetch(cc0 + 2, 0)
            compute_scatter(cc0 + 1, 1)
            return carry
        lax.fori_loop(0, NCH // 2, pair, 0)

        plsc.subcore_barrier()
        for i in range(NRT // EC):
            off = s * NRT + i * EC
            pltpu.sync_copy(acc_sh.at[pl.ds(off, EC)], av0)
            pltpu.sync_copy(av0, out_hbm.at[pl.ds(c * NP + off, EC)])

    return k


def _sc_deg_kernel():
    mesh = plsc.VectorSubcoreMesh(core_axis_name="c", subcore_axis_name="s")

    @functools.partial(
        pl.kernel,
        out_type=jax.ShapeDtypeStruct((2 * NP, 128), jnp.float32),
        mesh=mesh,
        scratch_types=[
            pltpu.VMEM((NCH, EC), jnp.int32),
            pltpu.VMEM((EC, 128), jnp.float32),
            pltpu.VMEM((EC, 128), jnp.float32),
            pltpu.VMEM_SHARED((NP, 128), jnp.float32),
            pltpu.SemaphoreType.DMA,
        ],
    )
    def k(dst_hbm, out_hbm, dst_all, ones_v, zero_v, acc_sh, sem):
        c = lax.axis_index("c")
        s = lax.axis_index("s")
        wid = c * 16 + s

        pltpu.sync_copy(dst_hbm.at[pl.ds(wid * NCH, NCH)], dst_all)
        one_row = jnp.where(lax.iota(jnp.int32, 16) == 0,
                            jnp.float32(1.0), jnp.float32(0.0))

        def fill(r, carry):
            ones_v[r, pl.ds(0, 16)] = one_row
            for j in range(1, 8):
                ones_v[r, pl.ds(16 * j, 16)] = jnp.zeros((16,), jnp.float32)
            for j in range(8):
                zero_v[r, pl.ds(16 * j, 16)] = jnp.zeros((16,), jnp.float32)
            return carry
        lax.fori_loop(0, EC, fill, 0)
        for i in range(NRT // EC):
            pltpu.sync_copy(zero_v, acc_sh.at[pl.ds(s * NRT + i * EC, EC)])
        plsc.subcore_barrier()

        nb = 8
        def batch(bb, carry):
            for t in range(nb):
                pltpu.async_copy(ones_v, acc_sh.at[dst_all.at[bb * nb + t]],
                                 sem, add=True)
            for t in range(nb):
                pltpu.make_async_copy(ones_v, acc_sh.at[dst_all.at[bb * nb + t]],
                                      sem).wait()
            return carry
        lax.fori_loop(0, NCH // nb, batch, 0)

        plsc.subcore_barrier()
        for i in range(NRT // EC):
            off = s * NRT + i * EC
            pltpu.sync_copy(acc_sh.at[pl.ds(off, EC)], zero_v)
            pltpu.sync_copy(zero_v, out_hbm.at[pl.ds(c * NP + off, EC)])

    return k


_EDGE_KERNEL = _sc_edge_kernel(H2P)
_DEG_KERNEL = _sc_deg_kernel()


def _gnn_fast(p, x_pad, eidx, ea_pad, deg, in_dim, hid, out_dim):
    dims = [(in_dim, hid, "conv1", "norm1"),
            (hid, hid // 2, "conv2", "norm2"),
            (hid // 2, out_dim, "conv3", "norm3")]
    f = tc_mlp4(
        x_pad,
        [p["fc1"]["w"].T, p["fc2"]["w"].T, p["fc3"]["w"].T, p["fc4"]["w"].T],
        [p["fc1"]["b"], p["fc2"]["b"], p["fc3"]["b"], p["fc4"]["b"]])
    h = x_pad
    for li, (di, h2, cname, nname) in enumerate(dims):
        l1 = p[cname + "_l1"]
        l2 = p[cname + "_l2"]
        w1 = l1["w"]
        pad = jnp.zeros((H2P, di), jnp.float32).at[:h2].set
        wd = pad(w1[:, :di]).T
        ws = pad(w1[:, di:2 * di]).T
        we = jnp.zeros((4, H2P), jnp.float32).at[:, :h2].set(w1[:, 2 * di:].T)
        b1 = jnp.zeros((H2P,), jnp.float32).at[:h2].set(l1["b"])
        a = tc_linear(h, wd, jnp.zeros((H2P,), jnp.float32))
        b = tc_linear(h, ws, jnp.zeros((H2P,), jnp.float32))
        s_parts = _EDGE_KERNEL(a, b, eidx, ea_pad, we, b1)
        s0 = s_parts[:NP]
        s1 = s_parts[NP:]
        nrm = p[nname]
        is_last = li == 2
        h = tc_postmp(s0, s1, deg, h2, l2["w"].T, l2["b"], nrm["g"],
                      nrm["b"], f=f if is_last else None)
    return h


def kernel(x, edge_index, edge_attr, enc_params, dec_params, koopman_blocks,
           sigma, L_w):
    n, in_dim = x.shape
    e = edge_attr.shape[0]
    hid = enc_params["fc1"]["w"].shape[0]
    koop = enc_params["fc4"]["w"].shape[0]
    num_obj, _, hh = sigma.shape
    m = koopman_blocks.shape[1]

    x_pad = jnp.zeros((NP, in_dim), jnp.float32).at[:n].set(x)
    dsti = jnp.full((EP,), NP - 1, jnp.int32).at[:e].set(
        edge_index[1].astype(jnp.int32)).reshape(EP // EC, EC)
    srci = jnp.zeros((EP,), jnp.int32).at[:e].set(
        edge_index[0].astype(jnp.int32)).reshape(EP // EC, EC)
    eidx = jnp.stack([dsti, srci], axis=1)
    ea_pad = jnp.zeros((EP, 4), jnp.float32).at[:e].set(edge_attr)
    ea_pad = ea_pad.reshape(EP * 4)

    deg_parts = _DEG_KERNEL(dsti)
    deg = (deg_parts[:NP, 0] + deg_parts[NP:, 0]).reshape(NP, 1)

    ks_pad = _gnn_fast(enc_params, x_pad, eidx, ea_pad, deg,
                       in_dim, hid, koop)

    dec_ae_pad = _gnn_fast(dec_params, ks_pad, eidx, ea_pad, deg,
                           koop, hid, in_dim)

    sigma2 = sigma.reshape(num_obj * num_obj, hh)
    koop2 = koopman_blocks.reshape(hh, m * m)
    kb2 = tc_mm_small(sigma2, koop2)
    kmat = kb2.reshape(num_obj, num_obj, m, m).transpose(0, 2, 1, 3)
    kmat = kmat.reshape(num_obj * m, num_obj * m)

    t_len = n
    u_pad = jnp.zeros((NP, 4), jnp.float32).at[:t_len - 1].set(
        edge_attr[:t_len - 1])
    c_in = tc_linear_bf16(u_pad, L_w.T, jnp.zeros((koop,), jnp.float32))
    gs = tc_scan_seq(ks_pad[0:8], c_in, kmat.astype(jnp.bfloat16), t_len)
    g_hat_pad = jnp.concatenate(
        [gs[:t_len], jnp.zeros((NP - t_len, koop), jnp.float32)], axis=0)

    dec_ro_pad = _gnn_fast(dec_params, g_hat_pad, eidx, ea_pad, deg,
                           koop, hid, in_dim)

    return (dec_ae_pad[:n], dec_ro_pad[:n], ks_pad[:n])


# speedup vs baseline: 5.6057x; 1.0055x over previous
import functools

import jax
import jax.numpy as jnp
from jax import lax
from jax.experimental import pallas as pl
from jax.experimental.pallas import tpu as pltpu
from jax.experimental.pallas import tpu_sc as plsc

_PH = jax.lax.Precision.HIGHEST

NP = 10240
BR = 1024
EP = 163840
EC = 80
TILES = 32
EPT = EP // TILES
NCH = EPT // EC
NRT = NP // 16


def _linear_body(x_ref, w_ref, b_ref, o_ref, *, act):
    y = jnp.dot(x_ref[...], w_ref[...], preferred_element_type=jnp.float32, precision=_PH)
    y = y + b_ref[0:1, :]
    if act == "relu":
        y = jnp.maximum(y, 0.0)
    o_ref[...] = y


def _linear_bf16_body(x_ref, w_ref, b_ref, o_ref):
    xb = x_ref[...].astype(jnp.bfloat16)
    wb = w_ref[...].astype(jnp.bfloat16)
    o_ref[...] = jnp.dot(xb, wb, preferred_element_type=jnp.float32) + b_ref[0:1, :]


def tc_linear_bf16(x, wt, b):
    n, di = x.shape
    do = wt.shape[1]
    b2d = jnp.tile(b.reshape(1, do), (8, 1))
    return pl.pallas_call(
        _linear_bf16_body,
        grid=(n // BR,),
        in_specs=[pl.BlockSpec((BR, di), lambda i: (i, 0)),
                  pl.BlockSpec((di, do), lambda i: (0, 0)),
                  pl.BlockSpec((8, do), lambda i: (0, 0))],
        out_specs=pl.BlockSpec((BR, do), lambda i: (i, 0)),
        out_shape=jax.ShapeDtypeStruct((n, do), jnp.float32),
    )(x, wt, b2d)


def tc_linear(x, wt, b, act="none"):
    n, di = x.shape
    do = wt.shape[1]
    b2d = jnp.tile(b.reshape(1, do), (8, 1))
    return pl.pallas_call(
        functools.partial(_linear_body, act=act),
        grid=(n // BR,),
        in_specs=[pl.BlockSpec((BR, di), lambda i: (i, 0)),
                  pl.BlockSpec((di, do), lambda i: (0, 0)),
                  pl.BlockSpec((8, do), lambda i: (0, 0))],
        out_specs=pl.BlockSpec((BR, do), lambda i: (i, 0)),
        out_shape=jax.ShapeDtypeStruct((n, do), jnp.float32),
    )(x, wt, b2d)


def _mlp4_body(x_ref, w1, b1, w2, b2, w3, b3, w4, b4, o_ref):
    h = jnp.maximum(jnp.dot(x_ref[...], w1[...],
                            preferred_element_type=jnp.float32, precision=_PH) + b1[0:1, :], 0.0)
    h = jnp.maximum(jnp.dot(h, w2[...],
                            preferred_element_type=jnp.float32, precision=_PH) + b2[0:1, :], 0.0)
    h = jnp.maximum(jnp.dot(h, w3[...],
                            preferred_element_type=jnp.float32, precision=_PH) + b3[0:1, :], 0.0)
    o_ref[...] = jnp.dot(h, w4[...],
                         preferred_element_type=jnp.float32, precision=_PH) + b4[0:1, :]


def tc_mlp4(x, ws, bs):
    n, di = x.shape
    do = ws[3].shape[1]
    args = [x]
    in_specs = [pl.BlockSpec((BR, di), lambda i: (i, 0))]
    for wt, b in zip(ws, bs):
        dwi, dwo = wt.shape
        args.append(wt)
        in_specs.append(pl.BlockSpec((dwi, dwo), lambda i: (0, 0)))
        args.append(jnp.tile(b.reshape(1, dwo), (8, 1)))
        in_specs.append(pl.BlockSpec((8, dwo), lambda i: (0, 0)))
    return pl.pallas_call(
        _mlp4_body,
        grid=(n // BR,),
        in_specs=in_specs,
        out_specs=pl.BlockSpec((BR, do), lambda i: (i, 0)),
        out_shape=jax.ShapeDtypeStruct((n, do), jnp.float32),
    )(*args)


def _postmp_body(s0_ref, s1_ref, deg_ref, w2_ref, b2_ref, g_ref, b_ref,
                 *rest, final, h2):
    if final:
        f_ref, o_ref = rest
    else:
        (o_ref,) = rest
    s = s0_ref[..., :h2] + s1_ref[..., :h2]
    y = jnp.dot(s, w2_ref[...], preferred_element_type=jnp.float32, precision=_PH)
    y = y + deg_ref[...] * b2_ref[0:1, :]
    y = jnp.maximum(y, 0.0)
    mu = jnp.mean(y, axis=1, keepdims=True)
    var = jnp.mean((y - mu) * (y - mu), axis=1, keepdims=True)
    y = (y - mu) / jnp.sqrt(var + 1e-5) * g_ref[0:1, :] + b_ref[0:1, :]
    if final:
        y = (y + f_ref[...]) * 0.5
    o_ref[...] = y


def tc_postmp(s0, s1, deg, h2, w2t, b2, ln_g, ln_b, f=None):
    n, wpad = s0.shape
    do = w2t.shape[1]
    final = f is not None
    args = [s0, s1, deg, w2t,
            jnp.tile(b2.reshape(1, do), (8, 1)),
            jnp.tile(ln_g.reshape(1, do), (8, 1)),
            jnp.tile(ln_b.reshape(1, do), (8, 1))]
    in_specs = [pl.BlockSpec((BR, wpad), lambda i: (i, 0)),
                pl.BlockSpec((BR, wpad), lambda i: (i, 0)),
                pl.BlockSpec((BR, 1), lambda i: (i, 0)),
                pl.BlockSpec((h2, do), lambda i: (0, 0)),
                pl.BlockSpec((8, do), lambda i: (0, 0)),
                pl.BlockSpec((8, do), lambda i: (0, 0)),
                pl.BlockSpec((8, do), lambda i: (0, 0))]
    if final:
        args.append(f)
        in_specs.append(pl.BlockSpec((BR, do), lambda i: (i, 0)))
    return pl.pallas_call(
        functools.partial(_postmp_body, final=final, h2=h2),
        grid=(n // BR,),
        in_specs=in_specs,
        out_specs=pl.BlockSpec((BR, do), lambda i: (i, 0)),
        out_shape=jax.ShapeDtypeStruct((n, do), jnp.float32),
    )(*args)


def _mm_body(a_ref, b_ref, o_ref):
    o_ref[...] = jnp.dot(a_ref[...], b_ref[...],
                         preferred_element_type=jnp.float32, precision=_PH)


def tc_mm_small(a, b):
    m, k = a.shape
    n = b.shape[1]
    return pl.pallas_call(
        _mm_body,
        in_specs=[pl.BlockSpec((m, k), lambda: (0, 0)),
                  pl.BlockSpec((k, n), lambda: (0, 0))],
        out_specs=pl.BlockSpec((m, n), lambda: (0, 0)),
        out_shape=jax.ShapeDtypeStruct((m, n), jnp.float32),
    )(a, b)


def _scan_seq_body(g0_ref, c_ref, kb_ref, o_ref, *, nblk):
    kbv = kb_ref[...]
    g = g0_ref[0:1, :]

    def blk(i, g):
        c8 = c_ref[pl.ds(8 * i, 8), :]
        rows = []
        for r in range(8):
            gb = g.astype(jnp.bfloat16)
            g = jnp.dot(gb, kbv, preferred_element_type=jnp.float32)
            g = g + c8[r:r + 1, :]
            rows.append(g)
        o_ref[pl.ds(8 * i, 8), :] = jnp.concatenate(rows, axis=0)
        return g
    lax.fori_loop(0, nblk, blk, g)


def tc_scan_seq(g0row, c, kmat_bf16, t_len):
    d = c.shape[1]
    nblk = (t_len + 7) // 8
    return pl.pallas_call(
        functools.partial(_scan_seq_body, nblk=nblk),
        in_specs=[pl.BlockSpec((8, d), lambda: (0, 0)),
                  pl.BlockSpec(c.shape, lambda: (0, 0)),
                  pl.BlockSpec((d, d), lambda: (0, 0))],
        out_specs=pl.BlockSpec(c.shape, lambda: (0, 0)),
        out_shape=jax.ShapeDtypeStruct(c.shape, jnp.float32),
    )(g0row, c, kmat_bf16)


H2P = 128


def _sc_edge_kernel(h2):
    nvr = h2 // 16
    mesh = plsc.VectorSubcoreMesh(core_axis_name="c", subcore_axis_name="s")

    @functools.partial(
        pl.kernel,
        out_type=jax.ShapeDtypeStruct((2 * NP, h2), jnp.float32),
        mesh=mesh,
        scratch_types=[
            pltpu.VMEM((2, EC), jnp.int32),
            pltpu.VMEM((2, EC), jnp.int32),
            pltpu.VMEM((EC * 4 + 16,), jnp.float32),
            pltpu.VMEM((EC * 4 + 16,), jnp.float32),
            pltpu.VMEM((EC, h2), jnp.float32),
            pltpu.VMEM((EC, h2), jnp.float32),
            pltpu.VMEM((EC, h2), jnp.float32),
            pltpu.VMEM((EC, h2), jnp.float32),
            pltpu.VMEM((4, h2), jnp.float32),
            pltpu.VMEM((h2,), jnp.float32),
            pltpu.VMEM_SHARED((NP, h2), jnp.float32),
            pltpu.SemaphoreType.DMA,
            pltpu.SemaphoreType.DMA,
            pltpu.SemaphoreType.DMA,
            pltpu.SemaphoreType.DMA,
            pltpu.SemaphoreType.DMA,
            pltpu.SemaphoreType.DMA,
        ],
    )
    def k(a_hbm, b_hbm, eidx_hbm, ea_hbm, wet_hbm, b1_hbm, out_hbm,
          idx0, idx1, ea0, ea1, av0, av1, bv0, bv1, wet_v, bias_v,
          acc_sh, se0, se1, sa0, sa1, sb0, sb1):
        c = lax.axis_index("c")
        s = lax.axis_index("s")
        wid = c * 16 + s
        idxs, eas, avs, bvs = [idx0, idx1], [ea0, ea1], [av0, av1], [bv0, bv1]
        sems_e, sems_a, sems_b = [se0, se1], [sa0, sa1], [sb0, sb1]

        pltpu.sync_copy(wet_hbm, wet_v)
        pltpu.sync_copy(b1_hbm, bias_v)

        def zb(r, carry):
            for j in range(nvr):
                av0[r, pl.ds(16 * j, 16)] = jnp.zeros((16,), jnp.float32)
            return carry
        lax.fori_loop(0, EC, zb, 0)
        for i in range(NRT // EC):
            pltpu.sync_copy(av0, acc_sh.at[pl.ds(s * NRT + i * EC, EC)])
        plsc.subcore_barrier()

        wvals = [[wet_v[d, pl.ds(16 * j, 16)] for j in range(nvr)]
                 for d in range(4)]
        bvals = [bias_v[pl.ds(16 * j, 16)] for j in range(nvr)]

        def fetch(cc, k):
            gcc = wid * NCH + cc
            base4 = (wid * EPT + cc * EC) * 4
            pltpu.sync_copy(eidx_hbm.at[gcc], idxs[k])
            pltpu.async_copy(ea_hbm.at[pl.ds(base4, EC * 4)],
                             eas[k].at[pl.ds(0, EC * 4)], sems_e[k])
            pltpu.async_copy(a_hbm.at[idxs[k].at[0]], avs[k], sems_a[k])
            pltpu.async_copy(b_hbm.at[idxs[k].at[1]], bvs[k], sems_b[k])

        def compute_scatter(cc, k):
            av, bv, eav = avs[k], bvs[k], eas[k]
            base4 = (wid * EPT + cc * EC) * 4
            pltpu.make_async_copy(ea_hbm.at[pl.ds(base4, EC * 4)],
                                  eav.at[pl.ds(0, EC * 4)], sems_e[k]).wait()
            pltpu.make_async_copy(a_hbm.at[idxs[k].at[0]], av,
                                  sems_a[k]).wait()
            pltpu.make_async_copy(b_hbm.at[idxs[k].at[1]], bv,
                                  sems_b[k]).wait()

            def edge(r, carry):
                ev = eav[pl.ds(4 * r, 16)]
                e0, e1, e2, e3 = ev[0], ev[1], ev[2], ev[3]
                for j in range(nvr):
                    sl = pl.ds(16 * j, 16)
                    v = av[r, sl] + bv[r, sl] + bvals[j]
                    v = v + e0 * wvals[0][j] + e1 * wvals[1][j]
                    v = v + e2 * wvals[2][j] + e3 * wvals[3][j]
                    av[r, sl] = jnp.maximum(v, 0.0)
                return carry
            lax.fori_loop(0, EC, edge, 0)
            pltpu.sync_copy(av, acc_sh.at[idxs[k].at[0]], add=True)

        fetch(0, 0)

        def pair(jj, carry):
            cc0 = 2 * jj
            fetch(cc0 + 1, 1)
            compute_scatter(cc0, 0)

            @pl.when(cc0 + 2 < NCH)
            def _():
                fetch(cc0 + 2, 0)
            compute_scatter(cc0 + 1, 1)
            return carry
        lax.fori_loop(0, NCH // 2, pair, 0)

        plsc.subcore_barrier()
        for i in range(NRT // EC):
            off = s * NRT + i * EC
            pltpu.sync_copy(acc_sh.at[pl.ds(off, EC)], av0)
            pltpu.sync_copy(av0, out_hbm.at[pl.ds(c * NP + off, EC)])

    return k


def _sc_deg_kernel():
    mesh = plsc.VectorSubcoreMesh(core_axis_name="c", subcore_axis_name="s")

    @functools.partial(
        pl.kernel,
        out_type=jax.ShapeDtypeStruct((2 * NP, 128), jnp.float32),
        mesh=mesh,
        scratch_types=[
            pltpu.VMEM((NCH, EC), jnp.int32),
            pltpu.VMEM((EC, 128), jnp.float32),
            pltpu.VMEM((EC, 128), jnp.float32),
            pltpu.VMEM_SHARED((NP, 128), jnp.float32),
            pltpu.SemaphoreType.DMA,
        ],
    )
    def k(dst_hbm, out_hbm, dst_all, ones_v, zero_v, acc_sh, sem):
        c = lax.axis_index("c")
        s = lax.axis_index("s")
        wid = c * 16 + s

        pltpu.sync_copy(dst_hbm.at[pl.ds(wid * NCH, NCH)], dst_all)
        one_row = jnp.where(lax.iota(jnp.int32, 16) == 0,
                            jnp.float32(1.0), jnp.float32(0.0))

        def fill(r, carry):
            ones_v[r, pl.ds(0, 16)] = one_row
            for j in range(1, 8):
                ones_v[r, pl.ds(16 * j, 16)] = jnp.zeros((16,), jnp.float32)
            for j in range(8):
                zero_v[r, pl.ds(16 * j, 16)] = jnp.zeros((16,), jnp.float32)
            return carry
        lax.fori_loop(0, EC, fill, 0)
        for i in range(NRT // EC):
            pltpu.sync_copy(zero_v, acc_sh.at[pl.ds(s * NRT + i * EC, EC)])
        plsc.subcore_barrier()

        nb = 8
        def batch(bb, carry):
            for t in range(nb):
                pltpu.async_copy(ones_v, acc_sh.at[dst_all.at[bb * nb + t]],
                                 sem, add=True)
            for t in range(nb):
                pltpu.make_async_copy(ones_v, acc_sh.at[dst_all.at[bb * nb + t]],
                                      sem).wait()
            return carry
        lax.fori_loop(0, NCH // nb, batch, 0)

        plsc.subcore_barrier()
        for i in range(NRT // EC):
            off = s * NRT + i * EC
            pltpu.sync_copy(acc_sh.at[pl.ds(off, EC)], zero_v)
            pltpu.sync_copy(zero_v, out_hbm.at[pl.ds(c * NP + off, EC)])

    return k


_EDGE_KERNEL = _sc_edge_kernel(H2P)
_DEG_KERNEL = _sc_deg_kernel()


def _gnn_fast(p, x_pad, eidx, ea_pad, deg, in_dim, hid, out_dim):
    dims = [(in_dim, hid, "conv1", "norm1"),
            (hid, hid // 2, "conv2", "norm2"),
            (hid // 2, out_dim, "conv3", "norm3")]
    f = tc_mlp4(
        x_pad,
        [p["fc1"]["w"].T, p["fc2"]["w"].T, p["fc3"]["w"].T, p["fc4"]["w"].T],
        [p["fc1"]["b"], p["fc2"]["b"], p["fc3"]["b"], p["fc4"]["b"]])
    h = x_pad
    for li, (di, h2, cname, nname) in enumerate(dims):
        l1 = p[cname + "_l1"]
        l2 = p[cname + "_l2"]
        w1 = l1["w"]
        pad = jnp.zeros((H2P, di), jnp.float32).at[:h2].set
        wd = pad(w1[:, :di]).T
        ws = pad(w1[:, di:2 * di]).T
        we = jnp.zeros((4, H2P), jnp.float32).at[:, :h2].set(w1[:, 2 * di:].T)
        b1 = jnp.zeros((H2P,), jnp.float32).at[:h2].set(l1["b"])
        a = tc_linear(h, wd, jnp.zeros((H2P,), jnp.float32))
        b = tc_linear(h, ws, jnp.zeros((H2P,), jnp.float32))
        s_parts = _EDGE_KERNEL(a, b, eidx, ea_pad, we, b1)
        s0 = s_parts[:NP]
        s1 = s_parts[NP:]
        nrm = p[nname]
        is_last = li == 2
        h = tc_postmp(s0, s1, deg, h2, l2["w"].T, l2["b"], nrm["g"],
                      nrm["b"], f=f if is_last else None)
    return h


def kernel(x, edge_index, edge_attr, enc_params, dec_params, koopman_blocks,
           sigma, L_w):
    n, in_dim = x.shape
    e = edge_attr.shape[0]
    hid = enc_params["fc1"]["w"].shape[0]
    koop = enc_params["fc4"]["w"].shape[0]
    num_obj, _, hh = sigma.shape
    m = koopman_blocks.shape[1]

    x_pad = jnp.zeros((NP, in_dim), jnp.float32).at[:n].set(x)
    dsti = jnp.full((EP,), NP - 1, jnp.int32).at[:e].set(
        edge_index[1].astype(jnp.int32)).reshape(EP // EC, EC)
    srci = jnp.zeros((EP,), jnp.int32).at[:e].set(
        edge_index[0].astype(jnp.int32)).reshape(EP // EC, EC)
    eidx = jnp.stack([dsti, srci], axis=1)
    ea_pad = jnp.zeros((EP, 4), jnp.float32).at[:e].set(edge_attr)
    ea_pad = ea_pad.reshape(EP * 4)

    deg_parts = _DEG_KERNEL(dsti)
    deg = (deg_parts[:NP, 0] + deg_parts[NP:, 0]).reshape(NP, 1)

    ks_pad = _gnn_fast(enc_params, x_pad, eidx, ea_pad, deg,
                       in_dim, hid, koop)

    dec_ae_pad = _gnn_fast(dec_params, ks_pad, eidx, ea_pad, deg,
                           koop, hid, in_dim)

    sigma2 = sigma.reshape(num_obj * num_obj, hh)
    koop2 = koopman_blocks.reshape(hh, m * m)
    kb2 = tc_mm_small(sigma2, koop2)
    kmat = kb2.reshape(num_obj, num_obj, m, m).transpose(0, 2, 1, 3)
    kmat = kmat.reshape(num_obj * m, num_obj * m)

    t_len = n
    u_pad = jnp.zeros((NP, 4), jnp.float32).at[:t_len - 1].set(
        edge_attr[:t_len - 1])
    c_in = tc_linear_bf16(u_pad, L_w.T, jnp.zeros((koop,), jnp.float32))
    gs = tc_scan_seq(ks_pad[0:8], c_in, kmat.astype(jnp.bfloat16), t_len)
    g_hat_pad = jnp.concatenate(
        [ks_pad[0:1], gs[:t_len - 1],
         jnp.zeros((NP - t_len, koop), jnp.float32)], axis=0)

    dec_ro_pad = _gnn_fast(dec_params, g_hat_pad, eidx, ea_pad, deg,
                           koop, hid, in_dim)

    return (dec_ae_pad[:n], dec_ro_pad[:n], ks_pad[:n])


# speedup vs baseline: 5.6058x; 1.0000x over previous
import functools

import jax
import jax.numpy as jnp
from jax import lax
from jax.experimental import pallas as pl
from jax.experimental.pallas import tpu as pltpu
from jax.experimental.pallas import tpu_sc as plsc

_PH = jax.lax.Precision.HIGHEST

NP = 10240
BR = 1024
EP = 163840
EC = 80
TILES = 32
EPT = EP // TILES
NCH = EPT // EC
NRT = NP // 16


def _linear_body(x_ref, w_ref, b_ref, o_ref, *, act):
    y = jnp.dot(x_ref[...], w_ref[...], preferred_element_type=jnp.float32, precision=_PH)
    y = y + b_ref[0:1, :]
    if act == "relu":
        y = jnp.maximum(y, 0.0)
    o_ref[...] = y


def _linear_bf16_body(x_ref, w_ref, b_ref, o_ref):
    xb = x_ref[...].astype(jnp.bfloat16)
    wb = w_ref[...].astype(jnp.bfloat16)
    o_ref[...] = jnp.dot(xb, wb, preferred_element_type=jnp.float32) + b_ref[0:1, :]


def tc_linear_bf16(x, wt, b):
    n, di = x.shape
    do = wt.shape[1]
    b2d = jnp.tile(b.reshape(1, do), (8, 1))
    return pl.pallas_call(
        _linear_bf16_body,
        grid=(n // BR,),
        in_specs=[pl.BlockSpec((BR, di), lambda i: (i, 0)),
                  pl.BlockSpec((di, do), lambda i: (0, 0)),
                  pl.BlockSpec((8, do), lambda i: (0, 0))],
        out_specs=pl.BlockSpec((BR, do), lambda i: (i, 0)),
        out_shape=jax.ShapeDtypeStruct((n, do), jnp.float32),
    )(x, wt, b2d)


def tc_linear(x, wt, b, act="none"):
    n, di = x.shape
    do = wt.shape[1]
    b2d = jnp.tile(b.reshape(1, do), (8, 1))
    return pl.pallas_call(
        functools.partial(_linear_body, act=act),
        grid=(n // BR,),
        in_specs=[pl.BlockSpec((BR, di), lambda i: (i, 0)),
                  pl.BlockSpec((di, do), lambda i: (0, 0)),
                  pl.BlockSpec((8, do), lambda i: (0, 0))],
        out_specs=pl.BlockSpec((BR, do), lambda i: (i, 0)),
        out_shape=jax.ShapeDtypeStruct((n, do), jnp.float32),
    )(x, wt, b2d)


def _mlp4_body(x_ref, w1, b1, w2, b2, w3, b3, w4, b4, o_ref):
    h = jnp.maximum(jnp.dot(x_ref[...], w1[...],
                            preferred_element_type=jnp.float32, precision=_PH) + b1[0:1, :], 0.0)
    h = jnp.maximum(jnp.dot(h, w2[...],
                            preferred_element_type=jnp.float32, precision=_PH) + b2[0:1, :], 0.0)
    h = jnp.maximum(jnp.dot(h, w3[...],
                            preferred_element_type=jnp.float32, precision=_PH) + b3[0:1, :], 0.0)
    o_ref[...] = jnp.dot(h, w4[...],
                         preferred_element_type=jnp.float32, precision=_PH) + b4[0:1, :]


def tc_mlp4(x, ws, bs):
    n, di = x.shape
    do = ws[3].shape[1]
    args = [x]
    in_specs = [pl.BlockSpec((BR, di), lambda i: (i, 0))]
    for wt, b in zip(ws, bs):
        dwi, dwo = wt.shape
        args.append(wt)
        in_specs.append(pl.BlockSpec((dwi, dwo), lambda i: (0, 0)))
        args.append(jnp.tile(b.reshape(1, dwo), (8, 1)))
        in_specs.append(pl.BlockSpec((8, dwo), lambda i: (0, 0)))
    return pl.pallas_call(
        _mlp4_body,
        grid=(n // BR,),
        in_specs=in_specs,
        out_specs=pl.BlockSpec((BR, do), lambda i: (i, 0)),
        out_shape=jax.ShapeDtypeStruct((n, do), jnp.float32),
    )(*args)


def _postmp_body(s0_ref, s1_ref, deg_ref, w2_ref, b2_ref, g_ref, b_ref,
                 *rest, final, h2):
    if final:
        f_ref, o_ref = rest
    else:
        (o_ref,) = rest
    s = s0_ref[..., :h2] + s1_ref[..., :h2]
    y = jnp.dot(s, w2_ref[...], preferred_element_type=jnp.float32, precision=_PH)
    y = y + deg_ref[...] * b2_ref[0:1, :]
    y = jnp.maximum(y, 0.0)
    mu = jnp.mean(y, axis=1, keepdims=True)
    var = jnp.mean((y - mu) * (y - mu), axis=1, keepdims=True)
    y = (y - mu) / jnp.sqrt(var + 1e-5) * g_ref[0:1, :] + b_ref[0:1, :]
    if final:
        y = (y + f_ref[...]) * 0.5
    o_ref[...] = y


def tc_postmp(s0, s1, deg, h2, w2t, b2, ln_g, ln_b, f=None):
    n, wpad = s0.shape
    do = w2t.shape[1]
    final = f is not None
    args = [s0, s1, deg, w2t,
            jnp.tile(b2.reshape(1, do), (8, 1)),
            jnp.tile(ln_g.reshape(1, do), (8, 1)),
            jnp.tile(ln_b.reshape(1, do), (8, 1))]
    in_specs = [pl.BlockSpec((BR, wpad), lambda i: (i, 0)),
                pl.BlockSpec((BR, wpad), lambda i: (i, 0)),
                pl.BlockSpec((BR, 1), lambda i: (i, 0)),
                pl.BlockSpec((h2, do), lambda i: (0, 0)),
                pl.BlockSpec((8, do), lambda i: (0, 0)),
                pl.BlockSpec((8, do), lambda i: (0, 0)),
                pl.BlockSpec((8, do), lambda i: (0, 0))]
    if final:
        args.append(f)
        in_specs.append(pl.BlockSpec((BR, do), lambda i: (i, 0)))
    return pl.pallas_call(
        functools.partial(_postmp_body, final=final, h2=h2),
        grid=(n // BR,),
        in_specs=in_specs,
        out_specs=pl.BlockSpec((BR, do), lambda i: (i, 0)),
        out_shape=jax.ShapeDtypeStruct((n, do), jnp.float32),
    )(*args)


def _mm_body(a_ref, b_ref, o_ref):
    o_ref[...] = jnp.dot(a_ref[...], b_ref[...],
                         preferred_element_type=jnp.float32, precision=_PH)


def tc_mm_small(a, b):
    m, k = a.shape
    n = b.shape[1]
    return pl.pallas_call(
        _mm_body,
        in_specs=[pl.BlockSpec((m, k), lambda: (0, 0)),
                  pl.BlockSpec((k, n), lambda: (0, 0))],
        out_specs=pl.BlockSpec((m, n), lambda: (0, 0)),
        out_shape=jax.ShapeDtypeStruct((m, n), jnp.float32),
    )(a, b)


def _scan_seq_body(g0_ref, c_ref, kb_ref, o_ref, *, nblk):
    kbv = kb_ref[...]
    g = g0_ref[0:1, :]

    def blk(i, g):
        c8 = c_ref[pl.ds(8 * i, 8), :]
        rows = []
        for r in range(8):
            gb = g.astype(jnp.bfloat16)
            g = jnp.dot(gb, kbv, preferred_element_type=jnp.float32)
            g = g + c8[r:r + 1, :]
            rows.append(g)
        o_ref[pl.ds(8 * i, 8), :] = jnp.concatenate(rows, axis=0)
        return g
    lax.fori_loop(0, nblk, blk, g)


def tc_scan_seq(g0row, c, kmat_bf16, t_len):
    d = c.shape[1]
    nblk = (t_len + 7) // 8
    return pl.pallas_call(
        functools.partial(_scan_seq_body, nblk=nblk),
        in_specs=[pl.BlockSpec((8, d), lambda: (0, 0)),
                  pl.BlockSpec(c.shape, lambda: (0, 0)),
                  pl.BlockSpec((d, d), lambda: (0, 0))],
        out_specs=pl.BlockSpec(c.shape, lambda: (0, 0)),
        out_shape=jax.ShapeDtypeStruct(c.shape, jnp.float32),
    )(g0row, c, kmat_bf16)


H2P = 128


def _sc_edge_kernel(h2):
    nvr = h2 // 16
    mesh = plsc.VectorSubcoreMesh(core_axis_name="c", subcore_axis_name="s")

    @functools.partial(
        pl.kernel,
        out_type=jax.ShapeDtypeStruct((2 * NP, h2), jnp.float32),
        mesh=mesh,
        scratch_types=[
            pltpu.VMEM((2, EC), jnp.int32),
            pltpu.VMEM((2, EC), jnp.int32),
            pltpu.VMEM((EC * 4 + 16,), jnp.float32),
            pltpu.VMEM((EC * 4 + 16,), jnp.float32),
            pltpu.VMEM((EC, h2), jnp.float32),
            pltpu.VMEM((EC, h2), jnp.float32),
            pltpu.VMEM((EC, h2), jnp.float32),
            pltpu.VMEM((EC, h2), jnp.float32),
            pltpu.VMEM((4, h2), jnp.float32),
            pltpu.VMEM((h2,), jnp.float32),
            pltpu.VMEM_SHARED((NP, h2), jnp.float32),
            pltpu.SemaphoreType.DMA,
            pltpu.SemaphoreType.DMA,
            pltpu.SemaphoreType.DMA,
            pltpu.SemaphoreType.DMA,
            pltpu.SemaphoreType.DMA,
            pltpu.SemaphoreType.DMA,
            pltpu.SemaphoreType.DMA,
            pltpu.SemaphoreType.DMA,
        ],
    )
    def k(a_hbm, b_hbm, eidx_hbm, ea_hbm, wet_hbm, b1_hbm, out_hbm,
          idx0, idx1, ea0, ea1, av0, av1, bv0, bv1, wet_v, bias_v,
          acc_sh, se0, se1, sa0, sa1, sb0, sb1, ss0, ss1):
        c = lax.axis_index("c")
        s = lax.axis_index("s")
        wid = c * 16 + s
        idxs, eas, avs, bvs = [idx0, idx1], [ea0, ea1], [av0, av1], [bv0, bv1]
        sems_e, sems_a, sems_b = [se0, se1], [sa0, sa1], [sb0, sb1]
        sems_s = [ss0, ss1]

        pltpu.sync_copy(wet_hbm, wet_v)
        pltpu.sync_copy(b1_hbm, bias_v)

        def zb(r, carry):
            for j in range(nvr):
                av0[r, pl.ds(16 * j, 16)] = jnp.zeros((16,), jnp.float32)
            return carry
        lax.fori_loop(0, EC, zb, 0)
        for i in range(NRT // EC):
            pltpu.sync_copy(av0, acc_sh.at[pl.ds(s * NRT + i * EC, EC)])
        plsc.subcore_barrier()

        wvals = [[wet_v[d, pl.ds(16 * j, 16)] for j in range(nvr)]
                 for d in range(4)]
        bvals = [bias_v[pl.ds(16 * j, 16)] for j in range(nvr)]

        def fetch(cc, k):
            @pl.when(cc >= 2)
            def _():
                pltpu.make_async_copy(avs[k], acc_sh.at[idxs[k].at[0]],
                                      sems_s[k]).wait()
            gcc = wid * NCH + cc
            base4 = (wid * EPT + cc * EC) * 4
            pltpu.sync_copy(eidx_hbm.at[gcc], idxs[k])
            pltpu.async_copy(ea_hbm.at[pl.ds(base4, EC * 4)],
                             eas[k].at[pl.ds(0, EC * 4)], sems_e[k])
            pltpu.async_copy(a_hbm.at[idxs[k].at[0]], avs[k], sems_a[k])
            pltpu.async_copy(b_hbm.at[idxs[k].at[1]], bvs[k], sems_b[k])

        def compute_scatter(cc, k):
            av, bv, eav = avs[k], bvs[k], eas[k]
            base4 = (wid * EPT + cc * EC) * 4
            pltpu.make_async_copy(ea_hbm.at[pl.ds(base4, EC * 4)],
                                  eav.at[pl.ds(0, EC * 4)], sems_e[k]).wait()
            pltpu.make_async_copy(a_hbm.at[idxs[k].at[0]], av,
                                  sems_a[k]).wait()
            pltpu.make_async_copy(b_hbm.at[idxs[k].at[1]], bv,
                                  sems_b[k]).wait()

            def edge(r, carry):
                ev = eav[pl.ds(4 * r, 16)]
                e0, e1, e2, e3 = ev[0], ev[1], ev[2], ev[3]
                for j in range(nvr):
                    sl = pl.ds(16 * j, 16)
                    v = av[r, sl] + bv[r, sl] + bvals[j]
                    v = v + e0 * wvals[0][j] + e1 * wvals[1][j]
                    v = v + e2 * wvals[2][j] + e3 * wvals[3][j]
                    av[r, sl] = jnp.maximum(v, 0.0)
                return carry
            lax.fori_loop(0, EC, edge, 0)
            pltpu.async_copy(av, acc_sh.at[idxs[k].at[0]], sems_s[k],
                             add=True)

        fetch(0, 0)

        def pair(jj, carry):
            cc0 = 2 * jj
            fetch(cc0 + 1, 1)
            compute_scatter(cc0, 0)

            @pl.when(cc0 + 2 < NCH)
            def _():
                fetch(cc0 + 2, 0)
            compute_scatter(cc0 + 1, 1)
            return carry
        lax.fori_loop(0, NCH // 2, pair, 0)
        pltpu.make_async_copy(av0, acc_sh.at[idx0.at[0]], ss0).wait()
        pltpu.make_async_copy(av1, acc_sh.at[idx1.at[0]], ss1).wait()

        plsc.subcore_barrier()
        for i in range(NRT // EC):
            off = s * NRT + i * EC
            pltpu.sync_copy(acc_sh.at[pl.ds(off, EC)], av0)
            pltpu.sync_copy(av0, out_hbm.at[pl.ds(c * NP + off, EC)])

    return k


def _sc_deg_kernel():
    mesh = plsc.VectorSubcoreMesh(core_axis_name="c", subcore_axis_name="s")

    @functools.partial(
        pl.kernel,
        out_type=jax.ShapeDtypeStruct((2 * NP, 128), jnp.float32),
        mesh=mesh,
        scratch_types=[
            pltpu.VMEM((NCH, EC), jnp.int32),
            pltpu.VMEM((EC, 128), jnp.float32),
            pltpu.VMEM((EC, 128), jnp.float32),
            pltpu.VMEM_SHARED((NP, 128), jnp.float32),
            pltpu.SemaphoreType.DMA,
        ],
    )
    def k(dst_hbm, out_hbm, dst_all, ones_v, zero_v, acc_sh, sem):
        c = lax.axis_index("c")
        s = lax.axis_index("s")
        wid = c * 16 + s

        pltpu.sync_copy(dst_hbm.at[pl.ds(wid * NCH, NCH)], dst_all)
        one_row = jnp.where(lax.iota(jnp.int32, 16) == 0,
                            jnp.float32(1.0), jnp.float32(0.0))

        def fill(r, carry):
            ones_v[r, pl.ds(0, 16)] = one_row
            for j in range(1, 8):
                ones_v[r, pl.ds(16 * j, 16)] = jnp.zeros((16,), jnp.float32)
            for j in range(8):
                zero_v[r, pl.ds(16 * j, 16)] = jnp.zeros((16,), jnp.float32)
            return carry
        lax.fori_loop(0, EC, fill, 0)
        for i in range(NRT // EC):
            pltpu.sync_copy(zero_v, acc_sh.at[pl.ds(s * NRT + i * EC, EC)])
        plsc.subcore_barrier()

        nb = 8
        def batch(bb, carry):
            for t in range(nb):
                pltpu.async_copy(ones_v, acc_sh.at[dst_all.at[bb * nb + t]],
                                 sem, add=True)
            for t in range(nb):
                pltpu.make_async_copy(ones_v, acc_sh.at[dst_all.at[bb * nb + t]],
                                      sem).wait()
            return carry
        lax.fori_loop(0, NCH // nb, batch, 0)

        plsc.subcore_barrier()
        for i in range(NRT // EC):
            off = s * NRT + i * EC
            pltpu.sync_copy(acc_sh.at[pl.ds(off, EC)], zero_v)
            pltpu.sync_copy(zero_v, out_hbm.at[pl.ds(c * NP + off, EC)])

    return k


_EDGE_KERNEL = _sc_edge_kernel(H2P)
_DEG_KERNEL = _sc_deg_kernel()


def _gnn_fast(p, x_pad, eidx, ea_pad, deg, in_dim, hid, out_dim):
    dims = [(in_dim, hid, "conv1", "norm1"),
            (hid, hid // 2, "conv2", "norm2"),
            (hid // 2, out_dim, "conv3", "norm3")]
    f = tc_mlp4(
        x_pad,
        [p["fc1"]["w"].T, p["fc2"]["w"].T, p["fc3"]["w"].T, p["fc4"]["w"].T],
        [p["fc1"]["b"], p["fc2"]["b"], p["fc3"]["b"], p["fc4"]["b"]])
    h = x_pad
    for li, (di, h2, cname, nname) in enumerate(dims):
        l1 = p[cname + "_l1"]
        l2 = p[cname + "_l2"]
        w1 = l1["w"]
        pad = jnp.zeros((H2P, di), jnp.float32).at[:h2].set
        wd = pad(w1[:, :di]).T
        ws = pad(w1[:, di:2 * di]).T
        we = jnp.zeros((4, H2P), jnp.float32).at[:, :h2].set(w1[:, 2 * di:].T)
        b1 = jnp.zeros((H2P,), jnp.float32).at[:h2].set(l1["b"])
        a = tc_linear(h, wd, jnp.zeros((H2P,), jnp.float32))
        b = tc_linear(h, ws, jnp.zeros((H2P,), jnp.float32))
        s_parts = _EDGE_KERNEL(a, b, eidx, ea_pad, we, b1)
        s0 = s_parts[:NP]
        s1 = s_parts[NP:]
        nrm = p[nname]
        is_last = li == 2
        h = tc_postmp(s0, s1, deg, h2, l2["w"].T, l2["b"], nrm["g"],
                      nrm["b"], f=f if is_last else None)
    return h


def kernel(x, edge_index, edge_attr, enc_params, dec_params, koopman_blocks,
           sigma, L_w):
    n, in_dim = x.shape
    e = edge_attr.shape[0]
    hid = enc_params["fc1"]["w"].shape[0]
    koop = enc_params["fc4"]["w"].shape[0]
    num_obj, _, hh = sigma.shape
    m = koopman_blocks.shape[1]

    x_pad = jnp.zeros((NP, in_dim), jnp.float32).at[:n].set(x)
    dsti = jnp.full((EP,), NP - 1, jnp.int32).at[:e].set(
        edge_index[1].astype(jnp.int32)).reshape(EP // EC, EC)
    srci = jnp.zeros((EP,), jnp.int32).at[:e].set(
        edge_index[0].astype(jnp.int32)).reshape(EP // EC, EC)
    eidx = jnp.stack([dsti, srci], axis=1)
    ea_pad = jnp.zeros((EP, 4), jnp.float32).at[:e].set(edge_attr)
    ea_pad = ea_pad.reshape(EP * 4)

    deg_parts = _DEG_KERNEL(dsti)
    deg = (deg_parts[:NP, 0] + deg_parts[NP:, 0]).reshape(NP, 1)

    ks_pad = _gnn_fast(enc_params, x_pad, eidx, ea_pad, deg,
                       in_dim, hid, koop)

    dec_ae_pad = _gnn_fast(dec_params, ks_pad, eidx, ea_pad, deg,
                           koop, hid, in_dim)

    sigma2 = sigma.reshape(num_obj * num_obj, hh)
    koop2 = koopman_blocks.reshape(hh, m * m)
    kb2 = tc_mm_small(sigma2, koop2)
    kmat = kb2.reshape(num_obj, num_obj, m, m).transpose(0, 2, 1, 3)
    kmat = kmat.reshape(num_obj * m, num_obj * m)

    t_len = n
    u_pad = jnp.zeros((NP, 4), jnp.float32).at[:t_len - 1].set(
        edge_attr[:t_len - 1])
    c_in = tc_linear_bf16(u_pad, L_w.T, jnp.zeros((koop,), jnp.float32))
    gs = tc_scan_seq(ks_pad[0:8], c_in, kmat.astype(jnp.bfloat16), t_len)
    g_hat_pad = jnp.concatenate(
        [ks_pad[0:1], gs[:t_len - 1],
         jnp.zeros((NP - t_len, koop), jnp.float32)], axis=0)

    dec_ro_pad = _gnn_fast(dec_params, g_hat_pad, eidx, ea_pad, deg,
                           koop, hid, in_dim)

    return (dec_ae_pad[:n], dec_ro_pad[:n], ks_pad[:n])


# speedup vs baseline: 5.6062x; 1.0001x over previous
import functools

import jax
import jax.numpy as jnp
from jax import lax
from jax.experimental import pallas as pl
from jax.experimental.pallas import tpu as pltpu
from jax.experimental.pallas import tpu_sc as plsc

_PH = jax.lax.Precision.HIGHEST

NP = 10240
BR = 1024
EP = 163840
EC = 80
TILES = 32
EPT = EP // TILES
NCH = EPT // EC
NRT = NP // 16


def _linear_body(x_ref, w_ref, b_ref, o_ref, *, act):
    y = jnp.dot(x_ref[...], w_ref[...], preferred_element_type=jnp.float32, precision=_PH)
    y = y + b_ref[0:1, :]
    if act == "relu":
        y = jnp.maximum(y, 0.0)
    o_ref[...] = y


def _linear_bf16_body(x_ref, w_ref, b_ref, o_ref):
    xb = x_ref[...].astype(jnp.bfloat16)
    wb = w_ref[...].astype(jnp.bfloat16)
    o_ref[...] = jnp.dot(xb, wb, preferred_element_type=jnp.float32) + b_ref[0:1, :]


def tc_linear_bf16(x, wt, b):
    n, di = x.shape
    do = wt.shape[1]
    b2d = jnp.tile(b.reshape(1, do), (8, 1))
    return pl.pallas_call(
        _linear_bf16_body,
        grid=(n // BR,),
        in_specs=[pl.BlockSpec((BR, di), lambda i: (i, 0)),
                  pl.BlockSpec((di, do), lambda i: (0, 0)),
                  pl.BlockSpec((8, do), lambda i: (0, 0))],
        out_specs=pl.BlockSpec((BR, do), lambda i: (i, 0)),
        out_shape=jax.ShapeDtypeStruct((n, do), jnp.float32),
    )(x, wt, b2d)


def tc_linear(x, wt, b, act="none"):
    n, di = x.shape
    do = wt.shape[1]
    b2d = jnp.tile(b.reshape(1, do), (8, 1))
    return pl.pallas_call(
        functools.partial(_linear_body, act=act),
        grid=(n // BR,),
        in_specs=[pl.BlockSpec((BR, di), lambda i: (i, 0)),
                  pl.BlockSpec((di, do), lambda i: (0, 0)),
                  pl.BlockSpec((8, do), lambda i: (0, 0))],
        out_specs=pl.BlockSpec((BR, do), lambda i: (i, 0)),
        out_shape=jax.ShapeDtypeStruct((n, do), jnp.float32),
    )(x, wt, b2d)


def _mlp4_body(x_ref, w1, b1, w2, b2, w3, b3, w4, b4, o_ref):
    h = jnp.maximum(jnp.dot(x_ref[...], w1[...],
                            preferred_element_type=jnp.float32, precision=_PH) + b1[0:1, :], 0.0)
    h = jnp.maximum(jnp.dot(h, w2[...],
                            preferred_element_type=jnp.float32, precision=_PH) + b2[0:1, :], 0.0)
    h = jnp.maximum(jnp.dot(h, w3[...],
                            preferred_element_type=jnp.float32, precision=_PH) + b3[0:1, :], 0.0)
    o_ref[...] = jnp.dot(h, w4[...],
                         preferred_element_type=jnp.float32, precision=_PH) + b4[0:1, :]


def tc_mlp4(x, ws, bs):
    n, di = x.shape
    do = ws[3].shape[1]
    args = [x]
    in_specs = [pl.BlockSpec((BR, di), lambda i: (i, 0))]
    for wt, b in zip(ws, bs):
        dwi, dwo = wt.shape
        args.append(wt)
        in_specs.append(pl.BlockSpec((dwi, dwo), lambda i: (0, 0)))
        args.append(jnp.tile(b.reshape(1, dwo), (8, 1)))
        in_specs.append(pl.BlockSpec((8, dwo), lambda i: (0, 0)))
    return pl.pallas_call(
        _mlp4_body,
        grid=(n // BR,),
        in_specs=in_specs,
        out_specs=pl.BlockSpec((BR, do), lambda i: (i, 0)),
        out_shape=jax.ShapeDtypeStruct((n, do), jnp.float32),
    )(*args)


def _postmp_body(s0_ref, s1_ref, deg_ref, w2_ref, b2_ref, g_ref, b_ref,
                 *rest, final, h2):
    if final:
        f_ref, o_ref = rest
    else:
        (o_ref,) = rest
    s = s0_ref[..., :h2] + s1_ref[..., :h2]
    y = jnp.dot(s, w2_ref[...], preferred_element_type=jnp.float32, precision=_PH)
    y = y + deg_ref[...] * b2_ref[0:1, :]
    y = jnp.maximum(y, 0.0)
    mu = jnp.mean(y, axis=1, keepdims=True)
    var = jnp.mean((y - mu) * (y - mu), axis=1, keepdims=True)
    y = (y - mu) / jnp.sqrt(var + 1e-5) * g_ref[0:1, :] + b_ref[0:1, :]
    if final:
        y = (y + f_ref[...]) * 0.5
    o_ref[...] = y


def tc_postmp(s0, s1, deg, h2, w2t, b2, ln_g, ln_b, f=None):
    n, wpad = s0.shape
    do = w2t.shape[1]
    final = f is not None
    args = [s0, s1, deg, w2t,
            jnp.tile(b2.reshape(1, do), (8, 1)),
            jnp.tile(ln_g.reshape(1, do), (8, 1)),
            jnp.tile(ln_b.reshape(1, do), (8, 1))]
    in_specs = [pl.BlockSpec((BR, wpad), lambda i: (i, 0)),
                pl.BlockSpec((BR, wpad), lambda i: (i, 0)),
                pl.BlockSpec((BR, 1), lambda i: (i, 0)),
                pl.BlockSpec((h2, do), lambda i: (0, 0)),
                pl.BlockSpec((8, do), lambda i: (0, 0)),
                pl.BlockSpec((8, do), lambda i: (0, 0)),
                pl.BlockSpec((8, do), lambda i: (0, 0))]
    if final:
        args.append(f)
        in_specs.append(pl.BlockSpec((BR, do), lambda i: (i, 0)))
    return pl.pallas_call(
        functools.partial(_postmp_body, final=final, h2=h2),
        grid=(n // BR,),
        in_specs=in_specs,
        out_specs=pl.BlockSpec((BR, do), lambda i: (i, 0)),
        out_shape=jax.ShapeDtypeStruct((n, do), jnp.float32),
    )(*args)


def _mm_body(a_ref, b_ref, o_ref):
    o_ref[...] = jnp.dot(a_ref[...], b_ref[...],
                         preferred_element_type=jnp.float32, precision=_PH)


def tc_mm_small(a, b):
    m, k = a.shape
    n = b.shape[1]
    return pl.pallas_call(
        _mm_body,
        in_specs=[pl.BlockSpec((m, k), lambda: (0, 0)),
                  pl.BlockSpec((k, n), lambda: (0, 0))],
        out_specs=pl.BlockSpec((m, n), lambda: (0, 0)),
        out_shape=jax.ShapeDtypeStruct((m, n), jnp.float32),
    )(a, b)


def _scan_seq_body(g0_ref, c_ref, kb_ref, o_ref, *, nblk):
    kbv = kb_ref[...]
    g = g0_ref[0:1, :]

    def blk(i, g):
        c8 = c_ref[pl.ds(8 * i, 8), :]
        rows = []
        for r in range(8):
            gb = g.astype(jnp.bfloat16)
            g = jnp.dot(gb, kbv, preferred_element_type=jnp.float32)
            g = g + c8[r:r + 1, :]
            rows.append(g)
        o_ref[pl.ds(8 * i, 8), :] = jnp.concatenate(rows, axis=0)
        return g
    lax.fori_loop(0, nblk, blk, g)


def tc_scan_seq(g0row, c, kmat_bf16, t_len):
    d = c.shape[1]
    nblk = (t_len + 7) // 8
    return pl.pallas_call(
        functools.partial(_scan_seq_body, nblk=nblk),
        in_specs=[pl.BlockSpec((8, d), lambda: (0, 0)),
                  pl.BlockSpec(c.shape, lambda: (0, 0)),
                  pl.BlockSpec((d, d), lambda: (0, 0))],
        out_specs=pl.BlockSpec(c.shape, lambda: (0, 0)),
        out_shape=jax.ShapeDtypeStruct(c.shape, jnp.float32),
    )(g0row, c, kmat_bf16)


H2P = 128


def _sc_edge_kernel(h2):
    nvr = h2 // 16
    mesh = plsc.VectorSubcoreMesh(core_axis_name="c", subcore_axis_name="s")

    @functools.partial(
        pl.kernel,
        out_type=jax.ShapeDtypeStruct((2 * NP, h2), jnp.float32),
        mesh=mesh,
        scratch_types=[
            pltpu.VMEM((2, EC), jnp.int32),
            pltpu.VMEM((2, EC), jnp.int32),
            pltpu.VMEM((EC * 4 + 16,), jnp.float32),
            pltpu.VMEM((EC * 4 + 16,), jnp.float32),
            pltpu.VMEM((EC, h2), jnp.float32),
            pltpu.VMEM((EC, h2), jnp.float32),
            pltpu.VMEM((EC, h2), jnp.float32),
            pltpu.VMEM((EC, h2), jnp.float32),
            pltpu.VMEM((4, h2), jnp.float32),
            pltpu.VMEM((h2,), jnp.float32),
            pltpu.VMEM_SHARED((NP, h2), jnp.float32),
            pltpu.SemaphoreType.DMA,
            pltpu.SemaphoreType.DMA,
            pltpu.SemaphoreType.DMA,
            pltpu.SemaphoreType.DMA,
            pltpu.SemaphoreType.DMA,
            pltpu.SemaphoreType.DMA,
        ],
    )
    def k(a_hbm, b_hbm, eidx_hbm, ea_hbm, wet_hbm, b1_hbm, out_hbm,
          idx0, idx1, ea0, ea1, av0, av1, bv0, bv1, wet_v, bias_v,
          acc_sh, se0, se1, sa0, sa1, sb0, sb1):
        c = lax.axis_index("c")
        s = lax.axis_index("s")
        wid = c * 16 + s
        idxs, eas, avs, bvs = [idx0, idx1], [ea0, ea1], [av0, av1], [bv0, bv1]
        sems_e, sems_a, sems_b = [se0, se1], [sa0, sa1], [sb0, sb1]

        pltpu.sync_copy(wet_hbm, wet_v)
        pltpu.sync_copy(b1_hbm, bias_v)

        def zb(r, carry):
            for j in range(nvr):
                av0[r, pl.ds(16 * j, 16)] = jnp.zeros((16,), jnp.float32)
            return carry
        lax.fori_loop(0, EC, zb, 0)
        for i in range(NRT // EC):
            pltpu.sync_copy(av0, acc_sh.at[pl.ds(s * NRT + i * EC, EC)])
        plsc.subcore_barrier()

        wvals = [[wet_v[d, pl.ds(16 * j, 16)] for j in range(nvr)]
                 for d in range(4)]
        bvals = [bias_v[pl.ds(16 * j, 16)] for j in range(nvr)]

        def fetch(cc, k):
            gcc = wid * NCH + cc
            base4 = (wid * EPT + cc * EC) * 4
            pltpu.sync_copy(eidx_hbm.at[gcc], idxs[k])
            pltpu.async_copy(ea_hbm.at[pl.ds(base4, EC * 4)],
                             eas[k].at[pl.ds(0, EC * 4)], sems_e[k])
            pltpu.async_copy(a_hbm.at[idxs[k].at[0]], avs[k], sems_a[k])
            pltpu.async_copy(b_hbm.at[idxs[k].at[1]], bvs[k], sems_b[k])

        def compute_scatter(cc, k):
            av, bv, eav = avs[k], bvs[k], eas[k]
            base4 = (wid * EPT + cc * EC) * 4
            pltpu.make_async_copy(ea_hbm.at[pl.ds(base4, EC * 4)],
                                  eav.at[pl.ds(0, EC * 4)], sems_e[k]).wait()
            pltpu.make_async_copy(a_hbm.at[idxs[k].at[0]], av,
                                  sems_a[k]).wait()
            pltpu.make_async_copy(b_hbm.at[idxs[k].at[1]], bv,
                                  sems_b[k]).wait()

            def edge(r, carry):
                ev = eav[pl.ds(4 * r, 16)]
                e0, e1, e2, e3 = ev[0], ev[1], ev[2], ev[3]
                for j in range(nvr):
                    sl = pl.ds(16 * j, 16)
                    v = av[r, sl] + bv[r, sl] + bvals[j]
                    v = v + e0 * wvals[0][j] + e1 * wvals[1][j]
                    v = v + e2 * wvals[2][j] + e3 * wvals[3][j]
                    av[r, sl] = jnp.maximum(v, 0.0)
                return carry
            lax.fori_loop(0, EC, edge, 0)
            pltpu.sync_copy(av, acc_sh.at[idxs[k].at[0]], add=True)

        fetch(0, 0)

        def pair(jj, carry):
            cc0 = 2 * jj
            fetch(cc0 + 1, 1)
            compute_scatter(cc0, 0)

            @pl.when(cc0 + 2 < NCH)
            def _():
                fetch(cc0 + 2, 0)
            compute_scatter(cc0 + 1, 1)
            return carry
        lax.fori_loop(0, NCH // 2, pair, 0)

        plsc.subcore_barrier()
        for i in range(NRT // EC):
            off = s * NRT + i * EC
            pltpu.sync_copy(acc_sh.at[pl.ds(off, EC)], av0)
            pltpu.sync_copy(av0, out_hbm.at[pl.ds(c * NP + off, EC)])

    return k


def _sc_deg_kernel():
    mesh = plsc.VectorSubcoreMesh(core_axis_name="c", subcore_axis_name="s")

    @functools.partial(
        pl.kernel,
        out_type=jax.ShapeDtypeStruct((2 * NP, 128), jnp.float32),
        mesh=mesh,
        scratch_types=[
            pltpu.VMEM((NCH, EC), jnp.int32),
            pltpu.VMEM((EC, 128), jnp.float32),
            pltpu.VMEM((EC, 128), jnp.float32),
            pltpu.VMEM_SHARED((NP, 128), jnp.float32),
            pltpu.SemaphoreType.DMA,
        ],
    )
    def k(dst_hbm, out_hbm, dst_all, ones_v, zero_v, acc_sh, sem):
        c = lax.axis_index("c")
        s = lax.axis_index("s")
        wid = c * 16 + s

        pltpu.sync_copy(dst_hbm.at[pl.ds(wid * NCH, NCH)], dst_all)
        one_row = jnp.where(lax.iota(jnp.int32, 16) == 0,
                            jnp.float32(1.0), jnp.float32(0.0))

        def fill(r, carry):
            ones_v[r, pl.ds(0, 16)] = one_row
            for j in range(1, 8):
                ones_v[r, pl.ds(16 * j, 16)] = jnp.zeros((16,), jnp.float32)
            for j in range(8):
                zero_v[r, pl.ds(16 * j, 16)] = jnp.zeros((16,), jnp.float32)
            return carry
        lax.fori_loop(0, EC, fill, 0)
        for i in range(NRT // EC):
            pltpu.sync_copy(zero_v, acc_sh.at[pl.ds(s * NRT + i * EC, EC)])
        plsc.subcore_barrier()

        nb = 8
        def batch(bb, carry):
            for t in range(nb):
                pltpu.async_copy(ones_v, acc_sh.at[dst_all.at[bb * nb + t]],
                                 sem, add=True)
            for t in range(nb):
                pltpu.make_async_copy(ones_v, acc_sh.at[dst_all.at[bb * nb + t]],
                                      sem).wait()
            return carry
        lax.fori_loop(0, NCH // nb, batch, 0)

        plsc.subcore_barrier()
        for i in range(NRT // EC):
            off = s * NRT + i * EC
            pltpu.sync_copy(acc_sh.at[pl.ds(off, EC)], zero_v)
            pltpu.sync_copy(zero_v, out_hbm.at[pl.ds(c * NP + off, EC)])

    return k


_EDGE_KERNEL = _sc_edge_kernel(H2P)
_DEG_KERNEL = _sc_deg_kernel()


def _gnn_fast(p, x_pad, eidx, ea_pad, deg, in_dim, hid, out_dim):
    dims = [(in_dim, hid, "conv1", "norm1"),
            (hid, hid // 2, "conv2", "norm2"),
            (hid // 2, out_dim, "conv3", "norm3")]
    f = tc_mlp4(
        x_pad,
        [p["fc1"]["w"].T, p["fc2"]["w"].T, p["fc3"]["w"].T, p["fc4"]["w"].T],
        [p["fc1"]["b"], p["fc2"]["b"], p["fc3"]["b"], p["fc4"]["b"]])
    h = x_pad
    for li, (di, h2, cname, nname) in enumerate(dims):
        l1 = p[cname + "_l1"]
        l2 = p[cname + "_l2"]
        w1 = l1["w"]
        pad = jnp.zeros((H2P, di), jnp.float32).at[:h2].set
        wd = pad(w1[:, :di]).T
        ws = pad(w1[:, di:2 * di]).T
        we = jnp.zeros((4, H2P), jnp.float32).at[:, :h2].set(w1[:, 2 * di:].T)
        b1 = jnp.zeros((H2P,), jnp.float32).at[:h2].set(l1["b"])
        a = tc_linear(h, wd, jnp.zeros((H2P,), jnp.float32))
        b = tc_linear(h, ws, jnp.zeros((H2P,), jnp.float32))
        s_parts = _EDGE_KERNEL(a, b, eidx, ea_pad, we, b1)
        s0 = s_parts[:NP]
        s1 = s_parts[NP:]
        nrm = p[nname]
        is_last = li == 2
        h = tc_postmp(s0, s1, deg, h2, l2["w"].T, l2["b"], nrm["g"],
                      nrm["b"], f=f if is_last else None)
    return h


def kernel(x, edge_index, edge_attr, enc_params, dec_params, koopman_blocks,
           sigma, L_w):
    n, in_dim = x.shape
    e = edge_attr.shape[0]
    hid = enc_params["fc1"]["w"].shape[0]
    koop = enc_params["fc4"]["w"].shape[0]
    num_obj, _, hh = sigma.shape
    m = koopman_blocks.shape[1]

    x_pad = jnp.zeros((NP, in_dim), jnp.float32).at[:n].set(x)
    dsti = jnp.full((EP,), NP - 1, jnp.int32).at[:e].set(
        edge_index[1].astype(jnp.int32)).reshape(EP // EC, EC)
    srci = jnp.zeros((EP,), jnp.int32).at[:e].set(
        edge_index[0].astype(jnp.int32)).reshape(EP // EC, EC)
    eidx = jnp.stack([dsti, srci], axis=1)
    ea_pad = jnp.zeros((EP, 4), jnp.float32).at[:e].set(edge_attr)
    ea_pad = ea_pad.reshape(EP * 4)

    deg_parts = _DEG_KERNEL(dsti)
    deg = (deg_parts[:NP, 0] + deg_parts[NP:, 0]).reshape(NP, 1)

    ks_pad = _gnn_fast(enc_params, x_pad, eidx, ea_pad, deg,
                       in_dim, hid, koop)

    dec_ae_pad = _gnn_fast(dec_params, ks_pad, eidx, ea_pad, deg,
                           koop, hid, in_dim)

    sigma2 = sigma.reshape(num_obj * num_obj, hh)
    koop2 = koopman_blocks.reshape(hh, m * m)
    kb2 = tc_mm_small(sigma2, koop2)
    kmat = kb2.reshape(num_obj, num_obj, m, m).transpose(0, 2, 1, 3)
    kmat = kmat.reshape(num_obj * m, num_obj * m)

    t_len = n
    u_pad = jnp.zeros((NP, 4), jnp.float32).at[:t_len - 1].set(
        edge_attr[:t_len - 1])
    c_in = tc_linear_bf16(u_pad, L_w.T, jnp.zeros((koop,), jnp.float32))
    gs = tc_scan_seq(ks_pad[0:8], c_in, kmat.astype(jnp.bfloat16), t_len)
    g_hat_pad = jnp.concatenate(
        [ks_pad[0:1], gs[:t_len - 1],
         jnp.zeros((NP - t_len, koop), jnp.float32)], axis=0)

    dec_ro_pad = _gnn_fast(dec_params, g_hat_pad, eidx, ea_pad, deg,
                           koop, hid, in_dim)

    return (dec_ae_pad[:n], dec_ro_pad[:n], ks_pad[:n])
